# Initial kernel scaffold; baseline (speedup 1.0000x reference)
import jax
import jax.numpy as jnp
from jax.experimental import pallas as pl


def kernel(item, video_features, u_h_embedding, uh_edge_index, v_uh_edge_index, trans_W, trans_b, gat1_W, gat1_att, sage2_W, sage2_b, gat3_W, gat3_att, sage4_W, sage4_b, uv_W, uv_b, uh_W, uh_b):
    raise NotImplementedError("write your pallas kernel here")


# speedup vs baseline: 1.1363x; 1.1363x over previous
import functools

import jax
import jax.numpy as jnp
from jax import lax
from jax.experimental import pallas as pl
from jax.experimental.pallas import tpu as pltpu

N_USER = 15000
N_UH = 25000
N_ALL = 50000
D = 64
B = 4096


def _leaky(x, s=0.01):
    return jnp.where(x >= 0, x, s * x)


def _head_body(user_ref, pos_ref, neg_ref, vid_ref, tW_ref, tb_ref,
               uvW_ref, uvb_ref, uhW_ref, uhb_ref, ps_ref, ns_ref):
    vid_t = _leaky(jnp.dot(vid_ref[...], tW_ref[...],
                           preferred_element_type=jnp.float32) + tb_ref[...])
    user = user_ref[...]
    usv = _leaky(jnp.dot(jnp.concatenate([vid_t, user], axis=1), uvW_ref[...],
                         preferred_element_type=jnp.float32) + uvb_ref[...])
    usph = _leaky(jnp.dot(jnp.concatenate([pos_ref[...], user], axis=1), uhW_ref[...],
                          preferred_element_type=jnp.float32) + uhb_ref[...])
    usnh = _leaky(jnp.dot(jnp.concatenate([neg_ref[...], user], axis=1), uhW_ref[...],
                          preferred_element_type=jnp.float32) + uhb_ref[...])
    ps_ref[...] = jnp.sum(usv * usph, axis=1, keepdims=True)
    ns_ref[...] = jnp.sum(usv * usnh, axis=1, keepdims=True)


def _head(user, pos_h, neg_h, vid, tW, tb, uvW, uvb, uhW, uhb):
    blk = 512
    grid = (B // blk,)
    row = lambda i: (i, 0)
    full = lambda i: (0, 0)
    ps, ns = pl.pallas_call(
        _head_body,
        grid=grid,
        in_specs=[
            pl.BlockSpec((blk, D), row),
            pl.BlockSpec((blk, D), row),
            pl.BlockSpec((blk, D), row),
            pl.BlockSpec((blk, 128), row),
            pl.BlockSpec((128, D), full),
            pl.BlockSpec((1, D), full),
            pl.BlockSpec((128, D), full),
            pl.BlockSpec((1, D), full),
            pl.BlockSpec((128, D), full),
            pl.BlockSpec((1, D), full),
        ],
        out_specs=[pl.BlockSpec((blk, 1), row), pl.BlockSpec((blk, 1), row)],
        out_shape=[jax.ShapeDtypeStruct((B, 1), jnp.float32),
                   jax.ShapeDtypeStruct((B, 1), jnp.float32)],
    )(user, pos_h, neg_h, vid, tW, tb.reshape(1, D), uvW, uvb.reshape(1, D),
      uhW, uhb.reshape(1, D))
    return ps.reshape(B), ns.reshape(B)


def kernel(item, video_features, u_h_embedding, uh_edge_index, v_uh_edge_index,
           trans_W, trans_b, gat1_W, gat1_att, sage2_W, sage2_b,
           gat3_W, gat3_att, sage4_W, sage4_b, uv_W, uv_b, uh_W, uh_b):
    vid_full = _leaky(video_features @ trans_W + trans_b)
    x = jnp.concatenate([u_h_embedding, vid_full], axis=0)
    x = x / jnp.maximum(jnp.linalg.norm(x, axis=1, keepdims=True), 1e-12)

    src_g, dst_g = v_uh_edge_index[0], v_uh_edge_index[1]
    src_s, dst_s = uh_edge_index[0], uh_edge_index[1]

    def gat(x, W, att):
        h = x @ W
        a_s = h @ att[:D]
        a_d = h @ att[D:]
        ub = _leaky(a_s.max() + a_d.max(), 0.2)
        w = jnp.exp(_leaky(a_s[src_g] + a_d[dst_g], 0.2) - ub)
        denom = jax.ops.segment_sum(w, dst_g, num_segments=N_ALL)
        acc = jax.ops.segment_sum(h[src_g] * w[:, None], dst_g, num_segments=N_ALL)
        return _leaky(acc / (denom[:, None] + 1e-16))

    cnt = jax.ops.segment_sum(jnp.ones(src_s.shape, jnp.float32), dst_s,
                              num_segments=N_ALL)

    def sage(x, W, b):
        s = jax.ops.segment_sum(x[src_s], dst_s, num_segments=N_ALL)
        mean = s / jnp.maximum(cnt, 1.0)[:, None]
        return _leaky(mean @ W + b)

    x1 = gat(x, gat1_W, gat1_att)
    x2 = sage(x1, sage2_W, sage2_b)
    x3 = gat(x2, gat3_W, gat3_att)
    x4 = sage(x3, sage4_W, sage4_b)
    result = x4[:N_UH]

    user = result[item[:, 0]]
    pos_h = result[item[:, 2]]
    neg_h = result[item[:, 3]]
    vid = vid_full[item[:, 1] - N_UH]
    vid_raw = video_features[item[:, 1] - N_UH]
    del vid
    return _head(user, pos_h, neg_h, vid_raw, trans_W, trans_b,
                 uv_W, uv_b, uh_W, uh_b)


# speedup vs baseline: 7.0850x; 6.2353x over previous
import functools

import jax
import jax.numpy as jnp
from jax import lax
from jax.experimental import pallas as pl
from jax.experimental.pallas import tpu as pltpu
from jax.experimental.pallas import tpu_sc as plsc

N_USER = 15000
N_UH = 25000
N_ALL = 50000
NP = 50176
D = 64
H = 32
B = 4096

E = 400000
EPT = 25600
E_PAD = EPT * 16
CH = 512
NCHUNK = EPT // CH
SUB = CH // 128
TRASH = N_ALL

BLK = 512
G = NP // BLK


def _leaky(x, s=0.01):
    return jnp.where(x >= 0, x, s * x)


def _x0_body(uh_ref, vid_ref, tW_ref, tb_ref, out_ref):
    r = pl.program_id(0)
    c = pl.program_id(2)

    @pl.when(r == 0)
    def _():
        row = uh_ref[...]
        nrm = jnp.maximum(jnp.sqrt(jnp.sum(row * row, axis=1, keepdims=True)), 1e-12)
        xn = row / nrm
        out_ref[0] = jnp.where(c == 0, xn[:, :H], xn[:, H:])

    @pl.when(r == 1)
    def _():
        v = _leaky(jnp.dot(vid_ref[...], tW_ref[...],
                           preferred_element_type=jnp.float32) + tb_ref[...])
        nrm = jnp.maximum(jnp.sqrt(jnp.sum(v * v, axis=1, keepdims=True)), 1e-12)
        xn = v / nrm
        out_ref[0] = jnp.where(c == 0, xn[:, :H], xn[:, H:])


def _build_x0(u_h_embedding, video_features, trans_W, trans_b):
    ab = 1000
    return pl.pallas_call(
        _x0_body,
        grid=(2, N_UH // ab, 2),
        in_specs=[
            pl.BlockSpec((ab, D), lambda r, i, c: (i, 0)),
            pl.BlockSpec((ab, 128), lambda r, i, c: (i, 0)),
            pl.BlockSpec((128, D), lambda r, i, c: (0, 0)),
            pl.BlockSpec((1, D), lambda r, i, c: (0, 0)),
        ],
        out_specs=pl.BlockSpec((1, ab, H), lambda r, i, c: (c, r * 25 + i, 0)),
        out_shape=jax.ShapeDtypeStruct((2, NP, H), jnp.float32),
    )(u_h_embedding, video_features, trans_W, trans_b.reshape(1, D))


def _gatprep_body(xa_ref, xb_ref, W_ref, att_ref, h_ref, as_ref, ad_ref,
                  mxs_ref, mxd_ref, *, from_sage, cnt_ref=None, sW_ref=None,
                  sb_ref=None):
    i = pl.program_id(0)
    c = pl.program_id(1)
    x = jnp.concatenate([xa_ref[0], xb_ref[0]], axis=1)
    if from_sage:
        mean = x / jnp.maximum(cnt_ref[...], 1.0)
        x = _leaky(jnp.dot(mean, sW_ref[...],
                           preferred_element_type=jnp.float32) + sb_ref[...])
    h = jnp.dot(x, W_ref[...], preferred_element_type=jnp.float32)
    h_ref[0] = jnp.where(c == 0, h[:, :H], h[:, H:])

    @pl.when(c == 0)
    def _():
        a_s = jnp.dot(h, att_ref[...][0, :], preferred_element_type=jnp.float32)
        a_d = jnp.dot(h, att_ref[...][1, :], preferred_element_type=jnp.float32)
        as_ref[...] = a_s[:, None]
        ad_ref[...] = a_d[:, None]
        rid = i * BLK + lax.broadcasted_iota(jnp.int32, (BLK, 1), 0)
        mask = rid < N_ALL
        ms = jnp.max(jnp.where(mask, a_s[:, None], -1e30), axis=(0, 1),
                     keepdims=True)
        md = jnp.max(jnp.where(mask, a_d[:, None], -1e30), axis=(0, 1),
                     keepdims=True)
        neg = jnp.full((1, 1), -1e30, jnp.float32)
        prev_s = jnp.where(i == 0, neg, mxs_ref[...])
        prev_d = jnp.where(i == 0, neg, mxd_ref[...])
        mxs_ref[...] = jnp.maximum(prev_s, ms)
        mxd_ref[...] = jnp.maximum(prev_d, md)


def _gat_prep(x_packed, W, att2):
    out = pl.pallas_call(
        functools.partial(_gatprep_body, from_sage=False),
        grid=(G, 2),
        in_specs=[
            pl.BlockSpec((1, BLK, H), lambda i, c: (0, i, 0)),
            pl.BlockSpec((1, BLK, H), lambda i, c: (1, i, 0)),
            pl.BlockSpec((D, D), lambda i, c: (0, 0)),
            pl.BlockSpec((2, D), lambda i, c: (0, 0)),
        ],
        out_specs=[
            pl.BlockSpec((1, BLK, H), lambda i, c: (c, i, 0)),
            pl.BlockSpec((BLK, 1), lambda i, c: (i, 0)),
            pl.BlockSpec((BLK, 1), lambda i, c: (i, 0)),
            pl.BlockSpec((1, 1), lambda i, c: (0, 0)),
            pl.BlockSpec((1, 1), lambda i, c: (0, 0)),
        ],
        out_shape=[
            jax.ShapeDtypeStruct((2, NP, H), jnp.float32),
            jax.ShapeDtypeStruct((NP, 1), jnp.float32),
            jax.ShapeDtypeStruct((NP, 1), jnp.float32),
            jax.ShapeDtypeStruct((1, 1), jnp.float32),
            jax.ShapeDtypeStruct((1, 1), jnp.float32),
        ],
    )(x_packed, x_packed, W, att2)
    return out


def _sage_gat_prep(s_packed, cnt, sW, sb, W, att2):
    def body(xa, xb, cnt_ref, sW_ref, sb_ref, W_ref, att_ref, h_ref, as_ref,
             ad_ref, mxs_ref, mxd_ref):
        _gatprep_body(xa, xb, W_ref, att_ref, h_ref, as_ref, ad_ref, mxs_ref,
                      mxd_ref, from_sage=True, cnt_ref=cnt_ref, sW_ref=sW_ref,
                      sb_ref=sb_ref)

    return pl.pallas_call(
        body,
        grid=(G, 2),
        in_specs=[
            pl.BlockSpec((1, BLK, H), lambda i, c: (0, i, 0)),
            pl.BlockSpec((1, BLK, H), lambda i, c: (1, i, 0)),
            pl.BlockSpec((BLK, 1), lambda i, c: (i, 0)),
            pl.BlockSpec((D, D), lambda i, c: (0, 0)),
            pl.BlockSpec((1, D), lambda i, c: (0, 0)),
            pl.BlockSpec((D, D), lambda i, c: (0, 0)),
            pl.BlockSpec((2, D), lambda i, c: (0, 0)),
        ],
        out_specs=[
            pl.BlockSpec((1, BLK, H), lambda i, c: (c, i, 0)),
            pl.BlockSpec((BLK, 1), lambda i, c: (i, 0)),
            pl.BlockSpec((BLK, 1), lambda i, c: (i, 0)),
            pl.BlockSpec((1, 1), lambda i, c: (0, 0)),
            pl.BlockSpec((1, 1), lambda i, c: (0, 0)),
        ],
        out_shape=[
            jax.ShapeDtypeStruct((2, NP, H), jnp.float32),
            jax.ShapeDtypeStruct((NP, 1), jnp.float32),
            jax.ShapeDtypeStruct((NP, 1), jnp.float32),
            jax.ShapeDtypeStruct((1, 1), jnp.float32),
            jax.ShapeDtypeStruct((1, 1), jnp.float32),
        ],
    )(s_packed, s_packed, cnt, sW, sb.reshape(1, D), W, att2)


def _divleaky_body(acc_ref, den_ref, x_ref):
    x_ref[0] = _leaky(acc_ref[0] / (den_ref[...] + 1e-16))


def _div_leaky(acc_packed, den):
    return pl.pallas_call(
        _divleaky_body,
        grid=(G, 2),
        in_specs=[
            pl.BlockSpec((1, BLK, H), lambda i, c: (c, i, 0)),
            pl.BlockSpec((BLK, 1), lambda i, c: (i, 0)),
        ],
        out_specs=pl.BlockSpec((1, BLK, H), lambda i, c: (c, i, 0)),
        out_shape=jax.ShapeDtypeStruct((2, NP, H), jnp.float32),
    )(acc_packed, den)


def _sage_final_body(sa_ref, sb2_ref, cnt_ref, W_ref, b_ref, out_ref):
    s = jnp.concatenate([sa_ref[0], sb2_ref[0]], axis=1)
    mean = s / jnp.maximum(cnt_ref[...], 1.0)
    out_ref[...] = _leaky(jnp.dot(mean, W_ref[...],
                                  preferred_element_type=jnp.float32) + b_ref[...])


def _sage_final(s_packed, cnt, W, b):
    return pl.pallas_call(
        _sage_final_body,
        grid=(49,),
        in_specs=[
            pl.BlockSpec((1, BLK, H), lambda i: (0, i, 0)),
            pl.BlockSpec((1, BLK, H), lambda i: (1, i, 0)),
            pl.BlockSpec((BLK, 1), lambda i: (i, 0)),
            pl.BlockSpec((D, D), lambda i: (0, 0)),
            pl.BlockSpec((1, D), lambda i: (0, 0)),
        ],
        out_specs=pl.BlockSpec((BLK, D), lambda i: (i, 0)),
        out_shape=jax.ShapeDtypeStruct((N_UH, D), jnp.float32),
    )(s_packed, s_packed, cnt, W, b.reshape(1, D))


def _head_body(user_ref, pos_ref, neg_ref, vid_ref, tW_ref, tb_ref,
               uvW_ref, uvb_ref, uhW_ref, uhb_ref, ps_ref, ns_ref):
    vid_t = _leaky(jnp.dot(vid_ref[...], tW_ref[...],
                           preferred_element_type=jnp.float32) + tb_ref[...])
    user = user_ref[...]
    usv = _leaky(jnp.dot(jnp.concatenate([vid_t, user], axis=1), uvW_ref[...],
                         preferred_element_type=jnp.float32) + uvb_ref[...])
    usph = _leaky(jnp.dot(jnp.concatenate([pos_ref[...], user], axis=1),
                          uhW_ref[...],
                          preferred_element_type=jnp.float32) + uhb_ref[...])
    usnh = _leaky(jnp.dot(jnp.concatenate([neg_ref[...], user], axis=1),
                          uhW_ref[...],
                          preferred_element_type=jnp.float32) + uhb_ref[...])
    ps_ref[...] = jnp.sum(usv * usph, axis=1, keepdims=True)
    ns_ref[...] = jnp.sum(usv * usnh, axis=1, keepdims=True)


def _head(upn, vid, tW, tb, uvW, uvb, uhW, uhb):
    row = lambda i: (i, 0)
    full = lambda i: (0, 0)
    ps, ns = pl.pallas_call(
        _head_body,
        grid=(B // BLK,),
        in_specs=[
            pl.BlockSpec((BLK, D), lambda i: (i, 0)),
            pl.BlockSpec((BLK, D), lambda i: (8 + i, 0)),
            pl.BlockSpec((BLK, D), lambda i: (16 + i, 0)),
            pl.BlockSpec((BLK, 128), row),
            pl.BlockSpec((128, D), full),
            pl.BlockSpec((1, D), full),
            pl.BlockSpec((128, D), full),
            pl.BlockSpec((1, D), full),
            pl.BlockSpec((128, D), full),
            pl.BlockSpec((1, D), full),
        ],
        out_specs=[pl.BlockSpec((BLK, 1), row), pl.BlockSpec((BLK, 1), row)],
        out_shape=[jax.ShapeDtypeStruct((B, 1), jnp.float32),
                   jax.ShapeDtypeStruct((B, 1), jnp.float32)],
    )(upn, upn, upn, vid, tW, tb.reshape(1, D), uvW, uvb.reshape(1, D),
      uhW, uhb.reshape(1, D))
    return ps.reshape(B), ns.reshape(B)


_TPR = NP // 16


def _zero_rows(rows_v, n):
    zero = jnp.zeros((16,), jnp.float32)

    def zb(i, _):
        rows_v[i, pl.ds(0, 16)] = zero
        rows_v[i, pl.ds(16, 16)] = zero
        return 0

    lax.fori_loop(0, n, zb, 0)


def _zero_1d(ref, n):
    zero = jnp.zeros((16,), jnp.float32)

    def zb(i, _):
        ref[pl.ds(i * 16, 16)] = zero
        return 0

    lax.fori_loop(0, n // 16, zb, 0)


_NFULL = _TPR // CH
_REM = _TPR % CH


def _zero_spmem(acc_sh, den_sh, rows_v, zero1_v, sid):
    base = sid * _TPR
    for q in range(_NFULL):
        pltpu.sync_copy(rows_v.at[pl.ds(0, CH)],
                        acc_sh.at[pl.ds(base + q * CH, CH)])
    pltpu.sync_copy(rows_v.at[pl.ds(0, _REM)],
                    acc_sh.at[pl.ds(base + _NFULL * CH, _REM)])
    if den_sh is not None:
        for q in range(_NFULL):
            pltpu.sync_copy(zero1_v.at[pl.ds(0, CH)],
                            den_sh.at[pl.ds(base + q * CH, CH)])
        pltpu.sync_copy(zero1_v.at[pl.ds(0, _REM)],
                        den_sh.at[pl.ds(base + _NFULL * CH, _REM)])


def _writeback(acc_sh, acc_hbm, rows_v, sid, out_base):
    base = sid * _TPR
    for q in range(_NFULL):
        pltpu.sync_copy(acc_sh.at[pl.ds(base + q * CH, CH)],
                        rows_v.at[pl.ds(0, CH)])
        pltpu.sync_copy(rows_v.at[pl.ds(0, CH)],
                        acc_hbm.at[pl.ds(out_base + base + q * CH, CH)])
    pltpu.sync_copy(acc_sh.at[pl.ds(base + _NFULL * CH, _REM)],
                    rows_v.at[pl.ds(0, _REM)])
    pltpu.sync_copy(rows_v.at[pl.ds(0, _REM)],
                    acc_hbm.at[pl.ds(out_base + base + _NFULL * CH, _REM)])


def _writeback_1d(den_sh, den_hbm, zero1_v, sid):
    base = sid * _TPR
    for q in range(_NFULL):
        pltpu.sync_copy(den_sh.at[pl.ds(base + q * CH, CH)],
                        zero1_v.at[pl.ds(0, CH)])
        pltpu.sync_copy(zero1_v.at[pl.ds(0, CH)],
                        den_hbm.at[pl.ds(base + q * CH, CH)])
    pltpu.sync_copy(den_sh.at[pl.ds(base + _NFULL * CH, _REM)],
                    zero1_v.at[pl.ds(0, _REM)])
    pltpu.sync_copy(zero1_v.at[pl.ds(0, _REM)],
                    den_hbm.at[pl.ds(base + _NFULL * CH, _REM)])


def _sc_gat_body(src_hbm, dst_hbm, as_hbm, ad_hbm, ub_hbm, h_hbm,
                 acc_hbm, den_hbm,
                 idx_v, dst_v, asg_v, adg_v, w_v, rows_v, ub_v, zero1_v,
                 acc_sh, den_sh, sem):
    cid = lax.axis_index("c")
    sid = lax.axis_index("s")

    _zero_rows(rows_v, CH)
    _zero_1d(zero1_v, CH)
    _zero_spmem(acc_sh, den_sh, rows_v, zero1_v, sid)
    pltpu.sync_copy(ub_hbm, ub_v)
    plsc.subcore_barrier()

    ubv = ub_v[...]
    off = cid * NP

    def chunk(k, _):
        row0 = sid * (EPT // 128) + k * SUB
        pltpu.sync_copy(src_hbm.at[pl.ds(row0, SUB)], idx_v)
        pltpu.sync_copy(dst_hbm.at[pl.ds(row0, SUB)], dst_v)
        cps = []
        for k2 in range(SUB):
            cps.append(pltpu.async_copy(as_hbm.at[idx_v.at[k2]],
                                        asg_v.at[k2], sem))
            cps.append(pltpu.async_copy(ad_hbm.at[dst_v.at[k2]],
                                        adg_v.at[k2], sem))
        for cp in cps:
            cp.wait()

        for k2 in range(SUB):
            def wbody(j, _):
                sl = pl.ds(j * 16, 16)
                a = asg_v[k2, sl] + adg_v[k2, sl]
                alpha = jnp.where(a >= 0, a, 0.2 * a)
                w_v[k2, sl] = jnp.exp(alpha - ubv)
                idx_v[k2, sl] = idx_v[k2, sl] + off
                return 0

            lax.fori_loop(0, 128 // 16, wbody, 0)

        cps = []
        for k2 in range(SUB):
            cps.append(pltpu.async_copy(h_hbm.at[idx_v.at[k2]],
                                        rows_v.at[pl.ds(k2 * 128, 128)], sem))
        for cp in cps:
            cp.wait()

        for k2 in range(SUB):
            def sbody(g, _):
                wv = w_v[k2, pl.ds(g * 16, 16)]
                for lane in range(16):
                    ws = wv[lane]
                    r = k2 * 128 + g * 16 + lane
                    rows_v[r, pl.ds(0, 16)] = rows_v[r, pl.ds(0, 16)] * ws
                    rows_v[r, pl.ds(16, 16)] = rows_v[r, pl.ds(16, 16)] * ws
                return 0

            lax.fori_loop(0, 128 // 16, sbody, 0)

        for k2 in range(SUB):
            pltpu.sync_copy(rows_v.at[pl.ds(k2 * 128, 128)],
                            acc_sh.at[dst_v.at[k2]], add=True)

        @pl.when(cid == 1)
        def _():
            for k2 in range(SUB):
                pltpu.sync_copy(w_v.at[k2], den_sh.at[dst_v.at[k2]], add=True)

        return 0

    lax.fori_loop(0, NCHUNK, chunk, 0)
    plsc.subcore_barrier()
    _writeback(acc_sh, acc_hbm, rows_v, sid, cid * NP)

    @pl.when(cid == 1)
    def _():
        _writeback_1d(den_sh, den_hbm, zero1_v, sid)


def _sc_sage_body(src_hbm, dst_hbm, x_hbm, *args, with_cnt):
    if with_cnt:
        (acc_hbm, cnt_hbm, idx_v, dst_v, ones_v, rows_v, zero1_v,
         acc_sh, den_sh, sem) = args
    else:
        (acc_hbm, idx_v, dst_v, ones_v, rows_v, zero1_v,
         acc_sh, den_sh, sem) = args
    cid = lax.axis_index("c")
    sid = lax.axis_index("s")

    _zero_rows(rows_v, CH)
    _zero_1d(zero1_v, CH)
    _zero_spmem(acc_sh, den_sh if with_cnt else None, rows_v, zero1_v, sid)
    if with_cnt:
        one = jnp.ones((16,), jnp.float32)
        for k2 in range(SUB):
            def ob(j, _):
                ones_v[k2, pl.ds(j * 16, 16)] = one
                return 0
            lax.fori_loop(0, 128 // 16, ob, 0)
    plsc.subcore_barrier()

    off = cid * NP

    def chunk(k, _):
        row0 = sid * (EPT // 128) + k * SUB
        pltpu.sync_copy(src_hbm.at[pl.ds(row0, SUB)], idx_v)
        pltpu.sync_copy(dst_hbm.at[pl.ds(row0, SUB)], dst_v)

        for k2 in range(SUB):
            def ibody(j, _):
                sl = pl.ds(j * 16, 16)
                idx_v[k2, sl] = idx_v[k2, sl] + off
                return 0

            lax.fori_loop(0, 128 // 16, ibody, 0)

        cps = []
        for k2 in range(SUB):
            cps.append(pltpu.async_copy(x_hbm.at[idx_v.at[k2]],
                                        rows_v.at[pl.ds(k2 * 128, 128)], sem))
        for cp in cps:
            cp.wait()

        for k2 in range(SUB):
            pltpu.sync_copy(rows_v.at[pl.ds(k2 * 128, 128)],
                            acc_sh.at[dst_v.at[k2]], add=True)

        if with_cnt:
            @pl.when(cid == 1)
            def _():
                for k2 in range(SUB):
                    pltpu.sync_copy(ones_v.at[k2], den_sh.at[dst_v.at[k2]],
                                    add=True)

        return 0

    lax.fori_loop(0, NCHUNK, chunk, 0)
    plsc.subcore_barrier()
    _writeback(acc_sh, acc_hbm, rows_v, sid, cid * NP)

    if with_cnt:
        @pl.when(cid == 1)
        def _():
            _writeback_1d(den_sh, cnt_hbm, zero1_v, sid)


def _sc_head_body(upn_idx_hbm, vid_idx_hbm, res_hbm, vidf_hbm,
                  upn_out, vid_out, idx_v, vidx_v, rows_v, vrow_v, sem):
    cid = lax.axis_index("c")
    sid = lax.axis_index("s")
    w = sid * 2 + cid

    pltpu.sync_copy(upn_idx_hbm.at[pl.ds(w * 3, 3)], idx_v)
    cps = []
    for k2 in range(3):
        cps.append(pltpu.async_copy(res_hbm.at[idx_v.at[k2]],
                                    rows_v.at[pl.ds(k2 * 128, 128)], sem))
    pltpu.sync_copy(vid_idx_hbm.at[pl.ds(w, 1)], vidx_v)
    cps.append(pltpu.async_copy(vidf_hbm.at[vidx_v.at[0]], vrow_v, sem))
    for cp in cps:
        cp.wait()
    pltpu.sync_copy(rows_v, upn_out.at[pl.ds(w * 384, 384)])
    pltpu.sync_copy(vrow_v, vid_out.at[pl.ds(w * 128, 128)])


def _sc_mesh():
    return plsc.VectorSubcoreMesh(core_axis_name="c", subcore_axis_name="s")


def _sc_gat(src2, dst2, a_s, a_d, ub16, h_flat):
    f32 = jnp.float32
    call = pl.kernel(
        _sc_gat_body,
        out_type=[
            jax.ShapeDtypeStruct((2 * NP, H), f32),
            jax.ShapeDtypeStruct((NP,), f32),
        ],
        mesh=_sc_mesh(),
        compiler_params=pltpu.CompilerParams(use_tc_tiling_on_sc=False),
        scratch_types=[
            pltpu.VMEM((SUB, 128), jnp.int32),
            pltpu.VMEM((SUB, 128), jnp.int32),
            pltpu.VMEM((SUB, 128), f32),
            pltpu.VMEM((SUB, 128), f32),
            pltpu.VMEM((SUB, 128), f32),
            pltpu.VMEM((CH, H), f32),
            pltpu.VMEM((16,), f32),
            pltpu.VMEM((CH,), f32),
            pltpu.VMEM_SHARED((NP, H), f32),
            pltpu.VMEM_SHARED((NP,), f32),
            pltpu.SemaphoreType.DMA,
        ],
    )
    return call(src2, dst2, a_s, a_d, ub16, h_flat)


def _sc_sage(src2, dst2, x_flat, with_cnt):
    f32 = jnp.float32
    out_type = [jax.ShapeDtypeStruct((2 * NP, H), f32)]
    if with_cnt:
        out_type.append(jax.ShapeDtypeStruct((NP,), f32))
    call = pl.kernel(
        functools.partial(_sc_sage_body, with_cnt=with_cnt),
        out_type=out_type,
        mesh=_sc_mesh(),
        compiler_params=pltpu.CompilerParams(use_tc_tiling_on_sc=False),
        scratch_types=[
            pltpu.VMEM((SUB, 128), jnp.int32),
            pltpu.VMEM((SUB, 128), jnp.int32),
            pltpu.VMEM((SUB, 128), f32),
            pltpu.VMEM((CH, H), f32),
            pltpu.VMEM((CH,), f32),
            pltpu.VMEM_SHARED((NP, H), f32),
            pltpu.VMEM_SHARED((NP,), f32),
            pltpu.SemaphoreType.DMA,
        ],
    )
    return call(src2, dst2, x_flat)


def _sc_head(upn_idx2, vid_idx2, result, video_features):
    f32 = jnp.float32
    call = pl.kernel(
        _sc_head_body,
        out_type=[
            jax.ShapeDtypeStruct((3 * B, D), f32),
            jax.ShapeDtypeStruct((B, 128), f32),
        ],
        mesh=_sc_mesh(),
        compiler_params=pltpu.CompilerParams(use_tc_tiling_on_sc=False),
        scratch_types=[
            pltpu.VMEM((3, 128), jnp.int32),
            pltpu.VMEM((1, 128), jnp.int32),
            pltpu.VMEM((384, D), f32),
            pltpu.VMEM((128, 128), f32),
            pltpu.SemaphoreType.DMA,
        ],
    )
    return call(upn_idx2, vid_idx2, result, video_features)


def _pad_edges(ei):
    pad = E_PAD - E
    src = jnp.concatenate([ei[0], jnp.zeros((pad,), jnp.int32)])
    dst = jnp.concatenate([ei[1], jnp.full((pad,), TRASH, jnp.int32)])
    return src.reshape(E_PAD // 128, 128), dst.reshape(E_PAD // 128, 128)


def kernel(item, video_features, u_h_embedding, uh_edge_index, v_uh_edge_index,
           trans_W, trans_b, gat1_W, gat1_att, sage2_W, sage2_b,
           gat3_W, gat3_att, sage4_W, sage4_b, uv_W, uv_b, uh_W, uh_b):
    f32 = jnp.float32
    src_g, dst_g = _pad_edges(v_uh_edge_index)
    src_s, dst_s = _pad_edges(uh_edge_index)

    x0 = _build_x0(u_h_embedding, video_features, trans_W, trans_b)

    att1 = jnp.stack([gat1_att[:D], gat1_att[D:]])
    att3 = jnp.stack([gat3_att[:D], gat3_att[D:]])

    h1, as1, ad1, mxs1, mxd1 = _gat_prep(x0, gat1_W, att1)
    ub1 = _leaky(mxs1[0, 0] + mxd1[0, 0], 0.2)
    acc1, den1 = _sc_gat(src_g, dst_g, as1.reshape(NP), ad1.reshape(NP),
                         jnp.full((16,), ub1, f32), h1.reshape(2 * NP, H))
    x1 = _div_leaky(acc1.reshape(2, NP, H), den1.reshape(NP, 1))

    s2, cnt = _sc_sage(src_s, dst_s, x1.reshape(2 * NP, H), with_cnt=True)
    cnt = cnt.reshape(NP, 1)

    h3, as3, ad3, mxs3, mxd3 = _sage_gat_prep(s2.reshape(2, NP, H), cnt,
                                              sage2_W, sage2_b, gat3_W, att3)
    ub3 = _leaky(mxs3[0, 0] + mxd3[0, 0], 0.2)
    acc3, den3 = _sc_gat(src_g, dst_g, as3.reshape(NP), ad3.reshape(NP),
                         jnp.full((16,), ub3, f32), h3.reshape(2 * NP, H))
    x3 = _div_leaky(acc3.reshape(2, NP, H), den3.reshape(NP, 1))

    (s4,) = _sc_sage(src_s, dst_s, x3.reshape(2 * NP, H), with_cnt=False)
    result = _sage_final(s4.reshape(2, NP, H), cnt, sage4_W, sage4_b)

    upn_idx = jnp.concatenate([item[:, 0], item[:, 2], item[:, 3]])
    upn_idx2 = upn_idx.reshape(3 * B // 128, 128)
    vid_idx2 = (item[:, 1] - N_UH).reshape(B // 128, 128)
    upn, vid = _sc_head(upn_idx2, vid_idx2, result, video_features)
    return _head(upn, vid, trans_W, trans_b, uv_W, uv_b, uh_W, uh_b)


# speedup vs baseline: 8.1005x; 1.1433x over previous
import functools

import jax
import jax.numpy as jnp
from jax import lax
from jax.experimental import pallas as pl
from jax.experimental.pallas import tpu as pltpu
from jax.experimental.pallas import tpu_sc as plsc

N_USER = 15000
N_UH = 25000
N_ALL = 50000
NP = 50176
D = 64
H = 32
B = 4096

E = 400000
EPT = 25600
E_PAD = EPT * 16
CB = 128
Q = 4
NBODY = EPT // (Q * CB)
TRASH = N_ALL

BLK = 512
G = NP // BLK
AB = 1000

_TPR = NP // 16
_NFULL = _TPR // CB
_REM = _TPR % CB


def _leaky(x, s=0.01):
    return jnp.where(x >= 0, x, s * x)


def _gat1prep_body(uh_ref, vid_ref, tW_ref, tb_ref, W_ref, att_ref,
                   h_ref, as_ref, ad_ref, mxs_ref, mxd_ref):
    r = pl.program_id(0)
    i = pl.program_id(1)
    c = pl.program_id(2)
    uh = uh_ref[...]
    nu = jnp.maximum(jnp.sqrt(jnp.sum(uh * uh, axis=1, keepdims=True)), 1e-12)
    xu = uh / nu
    v = _leaky(jnp.dot(vid_ref[...], tW_ref[...],
                       preferred_element_type=jnp.float32) + tb_ref[...])
    nv = jnp.maximum(jnp.sqrt(jnp.sum(v * v, axis=1, keepdims=True)), 1e-12)
    xv = v / nv
    x = jnp.where(r == 1, xv, xu)
    h = jnp.dot(x, W_ref[...], preferred_element_type=jnp.float32)
    h_ref[0] = jnp.where(c == 0, h[:, :H], h[:, H:])

    @pl.when(c == 0)
    def _():
        a_s = jnp.dot(h, att_ref[...][0, :], preferred_element_type=jnp.float32)
        a_d = jnp.dot(h, att_ref[...][1, :], preferred_element_type=jnp.float32)
        as_ref[...] = a_s[:, None]
        ad_ref[...] = a_d[:, None]
        ms = jnp.max(a_s[:, None], axis=(0, 1), keepdims=True)
        md = jnp.max(a_d[:, None], axis=(0, 1), keepdims=True)
        neg = jnp.full((1, 1), -1e30, jnp.float32)
        first = jnp.logical_and(r == 0, i == 0)
        prev_s = jnp.where(first, neg, mxs_ref[...])
        prev_d = jnp.where(first, neg, mxd_ref[...])
        mxs_ref[...] = jnp.maximum(prev_s, ms)
        mxd_ref[...] = jnp.maximum(prev_d, md)


def _gat1_prep(u_h_embedding, video_features, trans_W, trans_b, W, att2):
    full = lambda r, i, c: (0, 0)
    return pl.pallas_call(
        _gat1prep_body,
        grid=(2, N_UH // AB, 2),
        in_specs=[
            pl.BlockSpec((AB, D), lambda r, i, c: (i, 0)),
            pl.BlockSpec((AB, 128), lambda r, i, c: (i, 0)),
            pl.BlockSpec((128, D), full),
            pl.BlockSpec((1, D), full),
            pl.BlockSpec((D, D), full),
            pl.BlockSpec((2, D), full),
        ],
        out_specs=[
            pl.BlockSpec((1, AB, H), lambda r, i, c: (c, r * 25 + i, 0)),
            pl.BlockSpec((AB, 1), lambda r, i, c: (r * 25 + i, 0)),
            pl.BlockSpec((AB, 1), lambda r, i, c: (r * 25 + i, 0)),
            pl.BlockSpec((1, 1), full),
            pl.BlockSpec((1, 1), full),
        ],
        out_shape=[
            jax.ShapeDtypeStruct((2, NP, H), jnp.float32),
            jax.ShapeDtypeStruct((NP, 1), jnp.float32),
            jax.ShapeDtypeStruct((NP, 1), jnp.float32),
            jax.ShapeDtypeStruct((1, 1), jnp.float32),
            jax.ShapeDtypeStruct((1, 1), jnp.float32),
        ],
    )(u_h_embedding, video_features, trans_W, trans_b.reshape(1, D), W, att2)


def _sagegatprep_body(xa_ref, xb_ref, cnt_ref, sW_ref, sb_ref, W_ref, att_ref,
                      h_ref, as_ref, ad_ref, mxs_ref, mxd_ref):
    i = pl.program_id(0)
    c = pl.program_id(1)
    s = jnp.concatenate([xa_ref[0], xb_ref[0]], axis=1)
    mean = s / jnp.maximum(cnt_ref[...], 1.0)
    x = _leaky(jnp.dot(mean, sW_ref[...],
                       preferred_element_type=jnp.float32) + sb_ref[...])
    h = jnp.dot(x, W_ref[...], preferred_element_type=jnp.float32)
    h_ref[0] = jnp.where(c == 0, h[:, :H], h[:, H:])

    @pl.when(c == 0)
    def _():
        a_s = jnp.dot(h, att_ref[...][0, :], preferred_element_type=jnp.float32)
        a_d = jnp.dot(h, att_ref[...][1, :], preferred_element_type=jnp.float32)
        as_ref[...] = a_s[:, None]
        ad_ref[...] = a_d[:, None]
        rid = i * BLK + lax.broadcasted_iota(jnp.int32, (BLK, 1), 0)
        mask = rid < N_ALL
        ms = jnp.max(jnp.where(mask, a_s[:, None], -1e30), axis=(0, 1),
                     keepdims=True)
        md = jnp.max(jnp.where(mask, a_d[:, None], -1e30), axis=(0, 1),
                     keepdims=True)
        neg = jnp.full((1, 1), -1e30, jnp.float32)
        prev_s = jnp.where(i == 0, neg, mxs_ref[...])
        prev_d = jnp.where(i == 0, neg, mxd_ref[...])
        mxs_ref[...] = jnp.maximum(prev_s, ms)
        mxd_ref[...] = jnp.maximum(prev_d, md)


def _sage_gat_prep(s_packed, cnt, sW, sb, W, att2):
    full = lambda i, c: (0, 0)
    return pl.pallas_call(
        _sagegatprep_body,
        grid=(G, 2),
        in_specs=[
            pl.BlockSpec((1, BLK, H), lambda i, c: (0, i, 0)),
            pl.BlockSpec((1, BLK, H), lambda i, c: (1, i, 0)),
            pl.BlockSpec((BLK, 1), lambda i, c: (i, 0)),
            pl.BlockSpec((D, D), full),
            pl.BlockSpec((1, D), full),
            pl.BlockSpec((D, D), full),
            pl.BlockSpec((2, D), full),
        ],
        out_specs=[
            pl.BlockSpec((1, BLK, H), lambda i, c: (c, i, 0)),
            pl.BlockSpec((BLK, 1), lambda i, c: (i, 0)),
            pl.BlockSpec((BLK, 1), lambda i, c: (i, 0)),
            pl.BlockSpec((1, 1), full),
            pl.BlockSpec((1, 1), full),
        ],
        out_shape=[
            jax.ShapeDtypeStruct((2, NP, H), jnp.float32),
            jax.ShapeDtypeStruct((NP, 1), jnp.float32),
            jax.ShapeDtypeStruct((NP, 1), jnp.float32),
            jax.ShapeDtypeStruct((1, 1), jnp.float32),
            jax.ShapeDtypeStruct((1, 1), jnp.float32),
        ],
    )(s_packed, s_packed, cnt, sW, sb.reshape(1, D), W, att2)


def _divleaky_body(acc_ref, den_ref, x_ref):
    x_ref[0] = _leaky(acc_ref[0] / (den_ref[...] + 1e-16))


def _div_leaky(acc_packed, den):
    return pl.pallas_call(
        _divleaky_body,
        grid=(G, 2),
        in_specs=[
            pl.BlockSpec((1, BLK, H), lambda i, c: (c, i, 0)),
            pl.BlockSpec((BLK, 1), lambda i, c: (i, 0)),
        ],
        out_specs=pl.BlockSpec((1, BLK, H), lambda i, c: (c, i, 0)),
        out_shape=jax.ShapeDtypeStruct((2, NP, H), jnp.float32),
    )(acc_packed, den)


def _sage_final_body(sa_ref, sb2_ref, cnt_ref, W_ref, b_ref, out_ref):
    s = jnp.concatenate([sa_ref[0], sb2_ref[0]], axis=1)
    mean = s / jnp.maximum(cnt_ref[...], 1.0)
    out_ref[...] = _leaky(jnp.dot(mean, W_ref[...],
                                  preferred_element_type=jnp.float32) + b_ref[...])


def _sage_final(s_packed, cnt, W, b):
    return pl.pallas_call(
        _sage_final_body,
        grid=(49,),
        in_specs=[
            pl.BlockSpec((1, BLK, H), lambda i: (0, i, 0)),
            pl.BlockSpec((1, BLK, H), lambda i: (1, i, 0)),
            pl.BlockSpec((BLK, 1), lambda i: (i, 0)),
            pl.BlockSpec((D, D), lambda i: (0, 0)),
            pl.BlockSpec((1, D), lambda i: (0, 0)),
        ],
        out_specs=pl.BlockSpec((BLK, D), lambda i: (i, 0)),
        out_shape=jax.ShapeDtypeStruct((N_UH, D), jnp.float32),
    )(s_packed, s_packed, cnt, W, b.reshape(1, D))


def _head_body(user_ref, pos_ref, neg_ref, vid_ref, tW_ref, tb_ref,
               uvW_ref, uvb_ref, uhW_ref, uhb_ref, ps_ref, ns_ref):
    vid_t = _leaky(jnp.dot(vid_ref[...], tW_ref[...],
                           preferred_element_type=jnp.float32) + tb_ref[...])
    user = user_ref[...]
    usv = _leaky(jnp.dot(jnp.concatenate([vid_t, user], axis=1), uvW_ref[...],
                         preferred_element_type=jnp.float32) + uvb_ref[...])
    usph = _leaky(jnp.dot(jnp.concatenate([pos_ref[...], user], axis=1),
                          uhW_ref[...],
                          preferred_element_type=jnp.float32) + uhb_ref[...])
    usnh = _leaky(jnp.dot(jnp.concatenate([neg_ref[...], user], axis=1),
                          uhW_ref[...],
                          preferred_element_type=jnp.float32) + uhb_ref[...])
    ps_ref[...] = jnp.sum(usv * usph, axis=1, keepdims=True)
    ns_ref[...] = jnp.sum(usv * usnh, axis=1, keepdims=True)


def _head(upn, vid, tW, tb, uvW, uvb, uhW, uhb):
    row = lambda i: (i, 0)
    full = lambda i: (0, 0)
    ps, ns = pl.pallas_call(
        _head_body,
        grid=(B // BLK,),
        in_specs=[
            pl.BlockSpec((BLK, D), lambda i: (i, 0)),
            pl.BlockSpec((BLK, D), lambda i: (8 + i, 0)),
            pl.BlockSpec((BLK, D), lambda i: (16 + i, 0)),
            pl.BlockSpec((BLK, 128), row),
            pl.BlockSpec((128, D), full),
            pl.BlockSpec((1, D), full),
            pl.BlockSpec((128, D), full),
            pl.BlockSpec((1, D), full),
            pl.BlockSpec((128, D), full),
            pl.BlockSpec((1, D), full),
        ],
        out_specs=[pl.BlockSpec((BLK, 1), row), pl.BlockSpec((BLK, 1), row)],
        out_shape=[jax.ShapeDtypeStruct((B, 1), jnp.float32),
                   jax.ShapeDtypeStruct((B, 1), jnp.float32)],
    )(upn, upn, upn, vid, tW, tb.reshape(1, D), uvW, uvb.reshape(1, D),
      uhW, uhb.reshape(1, D))
    return ps.reshape(B), ns.reshape(B)


def _zero_rows(rows_v, n):
    zero = jnp.zeros((16,), jnp.float32)

    def zb(i, _):
        rows_v[i, pl.ds(0, 16)] = zero
        rows_v[i, pl.ds(16, 16)] = zero
        return 0

    lax.fori_loop(0, n, zb, 0)


def _zero_1d(ref, n):
    zero = jnp.zeros((16,), jnp.float32)

    def zb(i, _):
        ref[pl.ds(i * 16, 16)] = zero
        return 0

    lax.fori_loop(0, n // 16, zb, 0)


def _zero_spmem(acc_sh, den_sh, zrow_v, zero1_v, sid, sem):
    base = sid * _TPR
    hs = []
    for t in range(_NFULL):
        hs.append(pltpu.async_copy(zrow_v, acc_sh.at[pl.ds(base + t * CB, CB)],
                                   sem))
    hs.append(pltpu.async_copy(zrow_v.at[pl.ds(0, _REM)],
                               acc_sh.at[pl.ds(base + _NFULL * CB, _REM)], sem))
    if den_sh is not None:
        for t in range(_NFULL):
            hs.append(pltpu.async_copy(zero1_v,
                                       den_sh.at[pl.ds(base + t * CB, CB)], sem))
        hs.append(pltpu.async_copy(zero1_v.at[pl.ds(0, _REM)],
                                   den_sh.at[pl.ds(base + _NFULL * CB, _REM)],
                                   sem))
    for h_ in hs:
        h_.wait()


def _writeback(acc_sh, acc_hbm, rows, sid, out_base, sems):
    base = sid * _TPR
    hs = [None] * Q
    for t in range(_NFULL):
        b = t % Q
        if hs[b] is not None:
            hs[b].wait()
        pltpu.sync_copy(acc_sh.at[pl.ds(base + t * CB, CB)], rows[b])
        hs[b] = pltpu.async_copy(rows[b],
                                 acc_hbm.at[pl.ds(out_base + base + t * CB, CB)],
                                 sems[b])
    b = _NFULL % Q
    if hs[b] is not None:
        hs[b].wait()
        hs[b] = None
    pltpu.sync_copy(acc_sh.at[pl.ds(base + _NFULL * CB, _REM)],
                    rows[b].at[pl.ds(0, _REM)])
    hs[b] = pltpu.async_copy(rows[b].at[pl.ds(0, _REM)],
                             acc_hbm.at[pl.ds(out_base + base + _NFULL * CB,
                                              _REM)], sems[b])
    for h_ in hs:
        if h_ is not None:
            h_.wait()


def _writeback_1d(den_sh, den_hbm, zero1_v, sid, sem):
    base = sid * _TPR
    for t in range(_NFULL):
        pltpu.sync_copy(den_sh.at[pl.ds(base + t * CB, CB)], zero1_v)
        pltpu.sync_copy(zero1_v, den_hbm.at[pl.ds(base + t * CB, CB)])
    pltpu.sync_copy(den_sh.at[pl.ds(base + _NFULL * CB, _REM)],
                    zero1_v.at[pl.ds(0, _REM)])
    pltpu.sync_copy(zero1_v.at[pl.ds(0, _REM)],
                    den_hbm.at[pl.ds(base + _NFULL * CB, _REM)])


def _sc_gat_body(src_hbm, dst_hbm, as_hbm, ad_hbm, ub_hbm, h_hbm,
                 acc_hbm, den_hbm,
                 idxq, idxo, dstq, asg, adg, wq, r0, r1, r2, r3, ub_v, zero1_v,
                 acc_sh, den_sh,
                 sa0, sa1, sa2, sa3, sr0, sr1, sr2, sr3, ss):
    rows = (r0, r1, r2, r3)
    sas = (sa0, sa1, sa2, sa3)
    srs = (sr0, sr1, sr2, sr3)
    cid = lax.axis_index("c")
    sid = lax.axis_index("s")

    _zero_rows(r0, CB)
    _zero_1d(zero1_v, CB)
    _zero_spmem(acc_sh, den_sh, r0, zero1_v, sid, ss)
    pltpu.sync_copy(ub_hbm, ub_v)
    plsc.subcore_barrier()

    ubv = ub_v[...]
    off = cid * NP

    def body(j, _):
        row0 = sid * (EPT // CB) + j * Q
        pltpu.sync_copy(src_hbm.at[pl.ds(row0, Q)], idxq)
        pltpu.sync_copy(dst_hbm.at[pl.ds(row0, Q)], dstq)
        ghs = []
        for q in range(Q):
            ghs.append((pltpu.async_copy(as_hbm.at[idxq.at[q]], asg.at[q],
                                         sas[q]),
                        pltpu.async_copy(ad_hbm.at[dstq.at[q]], adg.at[q],
                                         sas[q])))
        for q in range(Q):
            def ib(g, _, q=q):
                sl = pl.ds(g * 16, 16)
                idxo[q, sl] = idxq[q, sl] + off
                return 0

            lax.fori_loop(0, CB // 16, ib, 0)
        rhs = []
        for q in range(Q):
            rhs.append(pltpu.async_copy(h_hbm.at[idxo.at[q]], rows[q], srs[q]))
        shs = []
        for q in range(Q):
            ghs[q][0].wait()
            ghs[q][1].wait()

            def wb(g, _, q=q):
                sl = pl.ds(g * 16, 16)
                a = asg[q, sl] + adg[q, sl]
                alpha = jnp.where(a >= 0, a, 0.2 * a)
                wq[q, sl] = jnp.exp(alpha - ubv)
                return 0

            lax.fori_loop(0, CB // 16, wb, 0)
            rhs[q].wait()

            def sb(g, _, q=q):
                wv = wq[q, pl.ds(g * 16, 16)]
                for lane in range(16):
                    ws = wv[lane]
                    rr = g * 16 + lane
                    rows[q][rr, pl.ds(0, 16)] = rows[q][rr, pl.ds(0, 16)] * ws
                    rows[q][rr, pl.ds(16, 16)] = rows[q][rr, pl.ds(16, 16)] * ws
                return 0

            lax.fori_loop(0, CB // 16, sb, 0)
            shs.append(pltpu.async_copy(rows[q], acc_sh.at[dstq.at[q]], ss,
                                        add=True))

        @pl.when(cid == 1)
        def _():
            for q in range(Q):
                pltpu.sync_copy(wq.at[q], den_sh.at[dstq.at[q]], add=True)

        for h_ in shs:
            h_.wait()
        return 0

    lax.fori_loop(0, NBODY, body, 0)
    plsc.subcore_barrier()
    _writeback(acc_sh, acc_hbm, rows, sid, cid * NP, srs)

    @pl.when(cid == 1)
    def _():
        _writeback_1d(den_sh, den_hbm, zero1_v, sid, ss)


def _sc_sage_body(src_hbm, dst_hbm, x_hbm, *args, with_cnt):
    if with_cnt:
        (acc_hbm, cnt_hbm, idxq, dstq, ones_v, r0, r1, r2, r3, zero1_v,
         acc_sh, den_sh, sr0, sr1, sr2, sr3, ss) = args
    else:
        (acc_hbm, idxq, dstq, ones_v, r0, r1, r2, r3, zero1_v,
         acc_sh, den_sh, sr0, sr1, sr2, sr3, ss) = args
    rows = (r0, r1, r2, r3)
    srs = (sr0, sr1, sr2, sr3)
    cid = lax.axis_index("c")
    sid = lax.axis_index("s")

    _zero_rows(r0, CB)
    _zero_1d(zero1_v, CB)
    _zero_spmem(acc_sh, den_sh if with_cnt else None, r0, zero1_v, sid, ss)
    if with_cnt:
        one = jnp.ones((16,), jnp.float32)
        for q in range(Q):
            def ob(g, _, q=q):
                ones_v[q, pl.ds(g * 16, 16)] = one
                return 0

            lax.fori_loop(0, CB // 16, ob, 0)
    plsc.subcore_barrier()

    off = cid * NP

    def body(j, _):
        row0 = sid * (EPT // CB) + j * Q
        pltpu.sync_copy(src_hbm.at[pl.ds(row0, Q)], idxq)
        pltpu.sync_copy(dst_hbm.at[pl.ds(row0, Q)], dstq)
        for q in range(Q):
            def ib(g, _, q=q):
                sl = pl.ds(g * 16, 16)
                idxq[q, sl] = idxq[q, sl] + off
                return 0

            lax.fori_loop(0, CB // 16, ib, 0)
        rhs = []
        for q in range(Q):
            rhs.append(pltpu.async_copy(x_hbm.at[idxq.at[q]], rows[q], srs[q]))
        shs = []
        for q in range(Q):
            rhs[q].wait()
            shs.append(pltpu.async_copy(rows[q], acc_sh.at[dstq.at[q]], ss,
                                        add=True))
        if with_cnt:
            @pl.when(cid == 1)
            def _():
                for q in range(Q):
                    pltpu.sync_copy(ones_v.at[q], den_sh.at[dstq.at[q]],
                                    add=True)
        for h_ in shs:
            h_.wait()
        return 0

    lax.fori_loop(0, NBODY, body, 0)
    plsc.subcore_barrier()
    _writeback(acc_sh, acc_hbm, rows, sid, cid * NP, srs)

    if with_cnt:
        @pl.when(cid == 1)
        def _():
            _writeback_1d(den_sh, cnt_hbm, zero1_v, sid, ss)


def _sc_head_body(upn_idx_hbm, vid_idx_hbm, res_hbm, vidf_hbm,
                  upn_out, vid_out, idx_v, vidx_v, rows_v, vrow_v, sem):
    cid = lax.axis_index("c")
    sid = lax.axis_index("s")
    w = sid * 2 + cid

    pltpu.sync_copy(upn_idx_hbm.at[pl.ds(w * 3, 3)], idx_v)
    cps = []
    for k2 in range(3):
        cps.append(pltpu.async_copy(res_hbm.at[idx_v.at[k2]],
                                    rows_v.at[pl.ds(k2 * 128, 128)], sem))
    pltpu.sync_copy(vid_idx_hbm.at[pl.ds(w, 1)], vidx_v)
    cps.append(pltpu.async_copy(vidf_hbm.at[vidx_v.at[0]], vrow_v, sem))
    for cp in cps:
        cp.wait()
    pltpu.sync_copy(rows_v, upn_out.at[pl.ds(w * 384, 384)])
    pltpu.sync_copy(vrow_v, vid_out.at[pl.ds(w * 128, 128)])


def _sc_mesh():
    return plsc.VectorSubcoreMesh(core_axis_name="c", subcore_axis_name="s")


def _sc_gat(src2, dst2, a_s, a_d, ub16, h_flat):
    f32 = jnp.float32
    call = pl.kernel(
        _sc_gat_body,
        out_type=[
            jax.ShapeDtypeStruct((2 * NP, H), f32),
            jax.ShapeDtypeStruct((NP,), f32),
        ],
        mesh=_sc_mesh(),
        compiler_params=pltpu.CompilerParams(use_tc_tiling_on_sc=False),
        scratch_types=[
            pltpu.VMEM((Q, CB), jnp.int32),
            pltpu.VMEM((Q, CB), jnp.int32),
            pltpu.VMEM((Q, CB), jnp.int32),
            pltpu.VMEM((Q, CB), f32),
            pltpu.VMEM((Q, CB), f32),
            pltpu.VMEM((Q, CB), f32),
            pltpu.VMEM((CB, H), f32),
            pltpu.VMEM((CB, H), f32),
            pltpu.VMEM((CB, H), f32),
            pltpu.VMEM((CB, H), f32),
            pltpu.VMEM((16,), f32),
            pltpu.VMEM((CB,), f32),
            pltpu.VMEM_SHARED((NP, H), f32),
            pltpu.VMEM_SHARED((NP,), f32),
            pltpu.SemaphoreType.DMA,
            pltpu.SemaphoreType.DMA,
            pltpu.SemaphoreType.DMA,
            pltpu.SemaphoreType.DMA,
            pltpu.SemaphoreType.DMA,
            pltpu.SemaphoreType.DMA,
            pltpu.SemaphoreType.DMA,
            pltpu.SemaphoreType.DMA,
            pltpu.SemaphoreType.DMA,
        ],
    )
    return call(src2, dst2, a_s, a_d, ub16, h_flat)


def _sc_sage(src2, dst2, x_flat, with_cnt):
    f32 = jnp.float32
    out_type = [jax.ShapeDtypeStruct((2 * NP, H), f32)]
    if with_cnt:
        out_type.append(jax.ShapeDtypeStruct((NP,), f32))
    call = pl.kernel(
        functools.partial(_sc_sage_body, with_cnt=with_cnt),
        out_type=out_type,
        mesh=_sc_mesh(),
        compiler_params=pltpu.CompilerParams(use_tc_tiling_on_sc=False),
        scratch_types=[
            pltpu.VMEM((Q, CB), jnp.int32),
            pltpu.VMEM((Q, CB), jnp.int32),
            pltpu.VMEM((Q, CB), f32),
            pltpu.VMEM((CB, H), f32),
            pltpu.VMEM((CB, H), f32),
            pltpu.VMEM((CB, H), f32),
            pltpu.VMEM((CB, H), f32),
            pltpu.VMEM((CB,), f32),
            pltpu.VMEM_SHARED((NP, H), f32),
            pltpu.VMEM_SHARED((NP,), f32),
            pltpu.SemaphoreType.DMA,
            pltpu.SemaphoreType.DMA,
            pltpu.SemaphoreType.DMA,
            pltpu.SemaphoreType.DMA,
            pltpu.SemaphoreType.DMA,
        ],
    )
    return call(src2, dst2, x_flat)


def _sc_head(upn_idx2, vid_idx2, result, video_features):
    f32 = jnp.float32
    call = pl.kernel(
        _sc_head_body,
        out_type=[
            jax.ShapeDtypeStruct((3 * B, D), f32),
            jax.ShapeDtypeStruct((B, 128), f32),
        ],
        mesh=_sc_mesh(),
        compiler_params=pltpu.CompilerParams(use_tc_tiling_on_sc=False),
        scratch_types=[
            pltpu.VMEM((3, 128), jnp.int32),
            pltpu.VMEM((1, 128), jnp.int32),
            pltpu.VMEM((384, D), f32),
            pltpu.VMEM((128, 128), f32),
            pltpu.SemaphoreType.DMA,
        ],
    )
    return call(upn_idx2, vid_idx2, result, video_features)


def _pad_edges(ei):
    pad = E_PAD - E
    src = jnp.concatenate([ei[0], jnp.zeros((pad,), jnp.int32)])
    dst = jnp.concatenate([ei[1], jnp.full((pad,), TRASH, jnp.int32)])
    return src.reshape(E_PAD // CB, CB), dst.reshape(E_PAD // CB, CB)


def kernel(item, video_features, u_h_embedding, uh_edge_index, v_uh_edge_index,
           trans_W, trans_b, gat1_W, gat1_att, sage2_W, sage2_b,
           gat3_W, gat3_att, sage4_W, sage4_b, uv_W, uv_b, uh_W, uh_b):
    f32 = jnp.float32
    src_g, dst_g = _pad_edges(v_uh_edge_index)
    src_s, dst_s = _pad_edges(uh_edge_index)

    att1 = jnp.stack([gat1_att[:D], gat1_att[D:]])
    att3 = jnp.stack([gat3_att[:D], gat3_att[D:]])

    h1, as1, ad1, mxs1, mxd1 = _gat1_prep(u_h_embedding, video_features,
                                          trans_W, trans_b, gat1_W, att1)
    ub1 = _leaky(mxs1[0, 0] + mxd1[0, 0], 0.2)
    acc1, den1 = _sc_gat(src_g, dst_g, as1.reshape(NP), ad1.reshape(NP),
                         jnp.full((16,), ub1, f32), h1.reshape(2 * NP, H))
    x1 = _div_leaky(acc1.reshape(2, NP, H), den1.reshape(NP, 1))

    s2, cnt = _sc_sage(src_s, dst_s, x1.reshape(2 * NP, H), with_cnt=True)
    cnt = cnt.reshape(NP, 1)

    h3, as3, ad3, mxs3, mxd3 = _sage_gat_prep(s2.reshape(2, NP, H), cnt,
                                              sage2_W, sage2_b, gat3_W, att3)
    ub3 = _leaky(mxs3[0, 0] + mxd3[0, 0], 0.2)
    acc3, den3 = _sc_gat(src_g, dst_g, as3.reshape(NP), ad3.reshape(NP),
                         jnp.full((16,), ub3, f32), h3.reshape(2 * NP, H))
    x3 = _div_leaky(acc3.reshape(2, NP, H), den3.reshape(NP, 1))

    (s4,) = _sc_sage(src_s, dst_s, x3.reshape(2 * NP, H), with_cnt=False)
    result = _sage_final(s4.reshape(2, NP, H), cnt, sage4_W, sage4_b)

    upn_idx = jnp.concatenate([item[:, 0], item[:, 2], item[:, 3]])
    upn_idx2 = upn_idx.reshape(3 * B // 128, 128)
    vid_idx2 = (item[:, 1] - N_UH).reshape(B // 128, 128)
    upn, vid = _sc_head(upn_idx2, vid_idx2, result, video_features)
    return _head(upn, vid, trans_W, trans_b, uv_W, uv_b, uh_W, uh_b)


# speedup vs baseline: 9.5031x; 1.1731x over previous
import functools

import jax
import jax.numpy as jnp
from jax import lax
from jax.experimental import pallas as pl
from jax.experimental.pallas import tpu as pltpu
from jax.experimental.pallas import tpu_sc as plsc

N_USER = 15000
N_UH = 25000
N_ALL = 50000
NP = 50176
D = 64
H = 32
B = 4096

E = 400000
EPT = 25600
E_PAD = EPT * 16
CB = 128
Q = 4
NBODY = EPT // (Q * CB)
TRASH = N_ALL

BLK = 512
BK = 7168
GK = NP // BK
AB = 5000

_TPR = NP // 16
_NFULL = _TPR // CB
_REM = _TPR % CB


def _leaky(x, s=0.01):
    return jnp.where(x >= 0, x, s * x)


def _gat1prep_body(uh_ref, vid_ref, tW_ref, tb_ref, W_ref, h_ref):
    r = pl.program_id(0)
    uh = uh_ref[...]
    nu = jnp.maximum(jnp.sqrt(jnp.sum(uh * uh, axis=1, keepdims=True)), 1e-12)
    xu = uh / nu
    v = _leaky(jnp.dot(vid_ref[...], tW_ref[...],
                       preferred_element_type=jnp.float32) + tb_ref[...])
    nv = jnp.maximum(jnp.sqrt(jnp.sum(v * v, axis=1, keepdims=True)), 1e-12)
    xv = v / nv
    x = jnp.where(r == 1, xv, xu)
    h = jnp.dot(x, W_ref[...], preferred_element_type=jnp.float32)
    c = pl.program_id(2)
    h_ref[0] = jnp.where(c == 0, h[:, :H], h[:, H:])


def _gat1_prep(u_h_embedding, video_features, trans_W, trans_b, W):
    full = lambda r, i, c: (0, 0)
    nb = N_UH // AB
    return pl.pallas_call(
        _gat1prep_body,
        grid=(2, nb, 2),
        in_specs=[
            pl.BlockSpec((AB, D), lambda r, i, c: (i, 0)),
            pl.BlockSpec((AB, 128), lambda r, i, c: (i, 0)),
            pl.BlockSpec((128, D), full),
            pl.BlockSpec((1, D), full),
            pl.BlockSpec((D, D), full),
        ],
        out_specs=pl.BlockSpec((1, AB, H),
                               lambda r, i, c: (c, r * (N_UH // AB) + i, 0)),
        out_shape=jax.ShapeDtypeStruct((2, NP, H), jnp.float32),
    )(u_h_embedding, video_features, trans_W, trans_b.reshape(1, D), W)


def _attn_body(ha_ref, hb_ref, att_ref, as_ref, ad_ref, mxs_ref, mxd_ref):
    i = pl.program_id(0)
    h = jnp.concatenate([ha_ref[0], hb_ref[0]], axis=1)
    a_s = jnp.dot(h, att_ref[...][0, :], preferred_element_type=jnp.float32)
    a_d = jnp.dot(h, att_ref[...][1, :], preferred_element_type=jnp.float32)
    as_ref[...] = a_s[:, None]
    ad_ref[...] = a_d[:, None]
    rid = i * BK + lax.broadcasted_iota(jnp.int32, (BK, 1), 0)
    mask = rid < N_ALL
    ms = jnp.max(jnp.where(mask, a_s[:, None], -1e30), axis=(0, 1),
                 keepdims=True)
    md = jnp.max(jnp.where(mask, a_d[:, None], -1e30), axis=(0, 1),
                 keepdims=True)
    neg = jnp.full((1, 1), -1e30, jnp.float32)
    prev_s = jnp.where(i == 0, neg, mxs_ref[...])
    prev_d = jnp.where(i == 0, neg, mxd_ref[...])
    mxs_ref[...] = jnp.maximum(prev_s, ms)
    mxd_ref[...] = jnp.maximum(prev_d, md)


def _attn_prep(h_packed, att2):
    full = lambda i: (0, 0)
    return pl.pallas_call(
        _attn_body,
        grid=(GK,),
        in_specs=[
            pl.BlockSpec((1, BK, H), lambda i: (0, i, 0)),
            pl.BlockSpec((1, BK, H), lambda i: (1, i, 0)),
            pl.BlockSpec((2, D), full),
        ],
        out_specs=[
            pl.BlockSpec((BK, 1), lambda i: (i, 0)),
            pl.BlockSpec((BK, 1), lambda i: (i, 0)),
            pl.BlockSpec((1, 1), full),
            pl.BlockSpec((1, 1), full),
        ],
        out_shape=[
            jax.ShapeDtypeStruct((NP, 1), jnp.float32),
            jax.ShapeDtypeStruct((NP, 1), jnp.float32),
            jax.ShapeDtypeStruct((1, 1), jnp.float32),
            jax.ShapeDtypeStruct((1, 1), jnp.float32),
        ],
    )(h_packed, h_packed, att2)


def _sagegatprep_body(xa_ref, xb_ref, cnt_ref, sW_ref, sb_ref, W_ref, att_ref,
                      h_ref, as_ref, ad_ref, mxs_ref, mxd_ref):
    i = pl.program_id(0)
    c = pl.program_id(1)
    s = jnp.concatenate([xa_ref[0], xb_ref[0]], axis=1)
    mean = s / jnp.maximum(cnt_ref[...], 1.0)
    x = _leaky(jnp.dot(mean, sW_ref[...],
                       preferred_element_type=jnp.float32) + sb_ref[...])
    h = jnp.dot(x, W_ref[...], preferred_element_type=jnp.float32)
    h_ref[0] = jnp.where(c == 0, h[:, :H], h[:, H:])

    @pl.when(c == 0)
    def _():
        a_s = jnp.dot(h, att_ref[...][0, :], preferred_element_type=jnp.float32)
        a_d = jnp.dot(h, att_ref[...][1, :], preferred_element_type=jnp.float32)
        as_ref[...] = a_s[:, None]
        ad_ref[...] = a_d[:, None]
        rid = i * BK + lax.broadcasted_iota(jnp.int32, (BK, 1), 0)
        mask = rid < N_ALL
        ms = jnp.max(jnp.where(mask, a_s[:, None], -1e30), axis=(0, 1),
                     keepdims=True)
        md = jnp.max(jnp.where(mask, a_d[:, None], -1e30), axis=(0, 1),
                     keepdims=True)
        neg = jnp.full((1, 1), -1e30, jnp.float32)
        prev_s = jnp.where(i == 0, neg, mxs_ref[...])
        prev_d = jnp.where(i == 0, neg, mxd_ref[...])
        mxs_ref[...] = jnp.maximum(prev_s, ms)
        mxd_ref[...] = jnp.maximum(prev_d, md)


def _sage_gat_prep(s_packed, cnt, sW, sb, W, att2):
    full = lambda i, c: (0, 0)
    return pl.pallas_call(
        _sagegatprep_body,
        grid=(GK, 2),
        in_specs=[
            pl.BlockSpec((1, BK, H), lambda i, c: (0, i, 0)),
            pl.BlockSpec((1, BK, H), lambda i, c: (1, i, 0)),
            pl.BlockSpec((BK, 1), lambda i, c: (i, 0)),
            pl.BlockSpec((D, D), full),
            pl.BlockSpec((1, D), full),
            pl.BlockSpec((D, D), full),
            pl.BlockSpec((2, D), full),
        ],
        out_specs=[
            pl.BlockSpec((1, BK, H), lambda i, c: (c, i, 0)),
            pl.BlockSpec((BK, 1), lambda i, c: (i, 0)),
            pl.BlockSpec((BK, 1), lambda i, c: (i, 0)),
            pl.BlockSpec((1, 1), full),
            pl.BlockSpec((1, 1), full),
        ],
        out_shape=[
            jax.ShapeDtypeStruct((2, NP, H), jnp.float32),
            jax.ShapeDtypeStruct((NP, 1), jnp.float32),
            jax.ShapeDtypeStruct((NP, 1), jnp.float32),
            jax.ShapeDtypeStruct((1, 1), jnp.float32),
            jax.ShapeDtypeStruct((1, 1), jnp.float32),
        ],
    )(s_packed, s_packed, cnt, sW, sb.reshape(1, D), W, att2)


def _divleaky_body(acc_ref, den_ref, x_ref):
    x_ref[0] = _leaky(acc_ref[0] / (den_ref[...] + 1e-16))


def _div_leaky(acc_packed, den):
    return pl.pallas_call(
        _divleaky_body,
        grid=(GK, 2),
        in_specs=[
            pl.BlockSpec((1, BK, H), lambda i, c: (c, i, 0)),
            pl.BlockSpec((BK, 1), lambda i, c: (i, 0)),
        ],
        out_specs=pl.BlockSpec((1, BK, H), lambda i, c: (c, i, 0)),
        out_shape=jax.ShapeDtypeStruct((2, NP, H), jnp.float32),
    )(acc_packed, den)


def _sage_final_body(sa_ref, sb2_ref, cnt_ref, W_ref, b_ref, out_ref):
    s = jnp.concatenate([sa_ref[0], sb2_ref[0]], axis=1)
    mean = s / jnp.maximum(cnt_ref[...], 1.0)
    out_ref[...] = _leaky(jnp.dot(mean, W_ref[...],
                                  preferred_element_type=jnp.float32) + b_ref[...])


def _sage_final(s_packed, cnt, W, b):
    fb = BK // 2
    return pl.pallas_call(
        _sage_final_body,
        grid=(-(-N_UH // fb),),
        in_specs=[
            pl.BlockSpec((1, fb, H), lambda i: (0, i, 0)),
            pl.BlockSpec((1, fb, H), lambda i: (1, i, 0)),
            pl.BlockSpec((fb, 1), lambda i: (i, 0)),
            pl.BlockSpec((D, D), lambda i: (0, 0)),
            pl.BlockSpec((1, D), lambda i: (0, 0)),
        ],
        out_specs=pl.BlockSpec((fb, D), lambda i: (i, 0)),
        out_shape=jax.ShapeDtypeStruct((N_UH, D), jnp.float32),
    )(s_packed, s_packed, cnt, W, b.reshape(1, D))


def _head_body(user_ref, pos_ref, neg_ref, vid_ref, tW_ref, tb_ref,
               uvW_ref, uvb_ref, uhW_ref, uhb_ref, ps_ref, ns_ref):
    vid_t = _leaky(jnp.dot(vid_ref[...], tW_ref[...],
                           preferred_element_type=jnp.float32) + tb_ref[...])
    user = user_ref[...]
    usv = _leaky(jnp.dot(jnp.concatenate([vid_t, user], axis=1), uvW_ref[...],
                         preferred_element_type=jnp.float32) + uvb_ref[...])
    usph = _leaky(jnp.dot(jnp.concatenate([pos_ref[...], user], axis=1),
                          uhW_ref[...],
                          preferred_element_type=jnp.float32) + uhb_ref[...])
    usnh = _leaky(jnp.dot(jnp.concatenate([neg_ref[...], user], axis=1),
                          uhW_ref[...],
                          preferred_element_type=jnp.float32) + uhb_ref[...])
    ps_ref[...] = jnp.sum(usv * usph, axis=1, keepdims=True)
    ns_ref[...] = jnp.sum(usv * usnh, axis=1, keepdims=True)


def _head(upn, vid, tW, tb, uvW, uvb, uhW, uhb):
    row = lambda i: (i, 0)
    full = lambda i: (0, 0)
    ps, ns = pl.pallas_call(
        _head_body,
        grid=(B // BLK,),
        in_specs=[
            pl.BlockSpec((BLK, D), lambda i: (i, 0)),
            pl.BlockSpec((BLK, D), lambda i: (8 + i, 0)),
            pl.BlockSpec((BLK, D), lambda i: (16 + i, 0)),
            pl.BlockSpec((BLK, 128), row),
            pl.BlockSpec((128, D), full),
            pl.BlockSpec((1, D), full),
            pl.BlockSpec((128, D), full),
            pl.BlockSpec((1, D), full),
            pl.BlockSpec((128, D), full),
            pl.BlockSpec((1, D), full),
        ],
        out_specs=[pl.BlockSpec((BLK, 1), row), pl.BlockSpec((BLK, 1), row)],
        out_shape=[jax.ShapeDtypeStruct((B, 1), jnp.float32),
                   jax.ShapeDtypeStruct((B, 1), jnp.float32)],
    )(upn, upn, upn, vid, tW, tb.reshape(1, D), uvW, uvb.reshape(1, D),
      uhW, uhb.reshape(1, D))
    return ps.reshape(B), ns.reshape(B)


def _zero_rows(rows_v, n):
    zero = jnp.zeros((16,), jnp.float32)

    def zb(i, _):
        rows_v[i, pl.ds(0, 16)] = zero
        rows_v[i, pl.ds(16, 16)] = zero
        return 0

    lax.fori_loop(0, n, zb, 0)


def _zero_1d(ref, n):
    zero = jnp.zeros((16,), jnp.float32)

    def zb(i, _):
        ref[pl.ds(i * 16, 16)] = zero
        return 0

    lax.fori_loop(0, n // 16, zb, 0)


def _zero_spmem(acc_sh, den_sh, zrow_v, zero1_v, sid, sem):
    base = sid * _TPR
    hs = []
    for t in range(_NFULL):
        hs.append(pltpu.async_copy(zrow_v, acc_sh.at[pl.ds(base + t * CB, CB)],
                                   sem))
    hs.append(pltpu.async_copy(zrow_v.at[pl.ds(0, _REM)],
                               acc_sh.at[pl.ds(base + _NFULL * CB, _REM)], sem))
    if den_sh is not None:
        for t in range(_NFULL):
            hs.append(pltpu.async_copy(zero1_v,
                                       den_sh.at[pl.ds(base + t * CB, CB)], sem))
        hs.append(pltpu.async_copy(zero1_v.at[pl.ds(0, _REM)],
                                   den_sh.at[pl.ds(base + _NFULL * CB, _REM)],
                                   sem))
    for h_ in hs:
        h_.wait()


def _writeback(acc_sh, acc_hbm, rows, sid, out_base, sems):
    base = sid * _TPR
    hs = [None] * Q
    for t in range(_NFULL):
        b = t % Q
        if hs[b] is not None:
            hs[b].wait()
        pltpu.sync_copy(acc_sh.at[pl.ds(base + t * CB, CB)], rows[b])
        hs[b] = pltpu.async_copy(rows[b],
                                 acc_hbm.at[pl.ds(out_base + base + t * CB, CB)],
                                 sems[b])
    b = _NFULL % Q
    if hs[b] is not None:
        hs[b].wait()
        hs[b] = None
    pltpu.sync_copy(acc_sh.at[pl.ds(base + _NFULL * CB, _REM)],
                    rows[b].at[pl.ds(0, _REM)])
    hs[b] = pltpu.async_copy(rows[b].at[pl.ds(0, _REM)],
                             acc_hbm.at[pl.ds(out_base + base + _NFULL * CB,
                                              _REM)], sems[b])
    for h_ in hs:
        if h_ is not None:
            h_.wait()


def _writeback_1d(den_sh, den_hbm, zero1_v, sid, sem):
    base = sid * _TPR
    for t in range(_NFULL):
        pltpu.sync_copy(den_sh.at[pl.ds(base + t * CB, CB)], zero1_v)
        pltpu.sync_copy(zero1_v, den_hbm.at[pl.ds(base + t * CB, CB)])
    pltpu.sync_copy(den_sh.at[pl.ds(base + _NFULL * CB, _REM)],
                    zero1_v.at[pl.ds(0, _REM)])
    pltpu.sync_copy(zero1_v.at[pl.ds(0, _REM)],
                    den_hbm.at[pl.ds(base + _NFULL * CB, _REM)])


def _sc_gat_body(src_hbm, dst_hbm, as_hbm, ad_hbm, ub_hbm, h_hbm,
                 acc_hbm, den_hbm,
                 idxq, idxo, dstq, asg, adg, wq, r0, r1, r2, r3, ub_v, zero1_v,
                 acc_sh, den_sh,
                 sa0, sa1, sa2, sa3, sr0, sr1, sr2, sr3, ss):
    rows = (r0, r1, r2, r3)
    sas = (sa0, sa1, sa2, sa3)
    srs = (sr0, sr1, sr2, sr3)
    cid = lax.axis_index("c")
    sid = lax.axis_index("s")

    _zero_rows(r0, CB)
    _zero_1d(zero1_v, CB)
    _zero_spmem(acc_sh, den_sh, r0, zero1_v, sid, ss)
    pltpu.sync_copy(ub_hbm, ub_v)
    plsc.subcore_barrier()

    ubv = ub_v[...]
    off = cid * NP

    def body(j, _):
        row0 = sid * (EPT // CB) + j * Q
        pltpu.sync_copy(src_hbm.at[pl.ds(row0, Q)], idxq)
        pltpu.sync_copy(dst_hbm.at[pl.ds(row0, Q)], dstq)
        ghs = []
        for q in range(Q):
            ghs.append((pltpu.async_copy(as_hbm.at[idxq.at[q]], asg.at[q],
                                         sas[q]),
                        pltpu.async_copy(ad_hbm.at[dstq.at[q]], adg.at[q],
                                         sas[q])))
        for q in range(Q):
            def ib(g, _, q=q):
                sl = pl.ds(g * 16, 16)
                idxo[q, sl] = idxq[q, sl] + off
                return 0

            lax.fori_loop(0, CB // 16, ib, 0)
        rhs = []
        for q in range(Q):
            rhs.append(pltpu.async_copy(h_hbm.at[idxo.at[q]], rows[q], srs[q]))
        shs = []
        for q in range(Q):
            ghs[q][0].wait()
            ghs[q][1].wait()

            def wb(g, _, q=q):
                sl = pl.ds(g * 16, 16)
                a = asg[q, sl] + adg[q, sl]
                alpha = jnp.where(a >= 0, a, 0.2 * a)
                wq[q, sl] = jnp.exp(alpha - ubv)
                return 0

            lax.fori_loop(0, CB // 16, wb, 0)
            rhs[q].wait()

            def sb(g, _, q=q):
                wv = wq[q, pl.ds(g * 16, 16)]
                for lane in range(16):
                    ws = wv[lane]
                    rr = g * 16 + lane
                    rows[q][rr, pl.ds(0, 16)] = rows[q][rr, pl.ds(0, 16)] * ws
                    rows[q][rr, pl.ds(16, 16)] = rows[q][rr, pl.ds(16, 16)] * ws
                return 0

            lax.fori_loop(0, CB // 16, sb, 0)
            shs.append(pltpu.async_copy(rows[q], acc_sh.at[dstq.at[q]], ss,
                                        add=True))

        @pl.when(cid == 1)
        def _():
            for q in range(Q):
                pltpu.sync_copy(wq.at[q], den_sh.at[dstq.at[q]], add=True)

        for h_ in shs:
            h_.wait()
        return 0

    lax.fori_loop(0, NBODY, body, 0)
    plsc.subcore_barrier()
    _writeback(acc_sh, acc_hbm, rows, sid, cid * NP, srs)

    @pl.when(cid == 1)
    def _():
        _writeback_1d(den_sh, den_hbm, zero1_v, sid, ss)


def _sc_sage_body(src_hbm, dst_hbm, x_hbm, *args, with_cnt):
    if with_cnt:
        (acc_hbm, cnt_hbm, idxq, dstq, ones_v, r0, r1, r2, r3, zero1_v,
         acc_sh, den_sh, sr0, sr1, sr2, sr3, ss) = args
    else:
        (acc_hbm, idxq, dstq, ones_v, r0, r1, r2, r3, zero1_v,
         acc_sh, den_sh, sr0, sr1, sr2, sr3, ss) = args
    rows = (r0, r1, r2, r3)
    srs = (sr0, sr1, sr2, sr3)
    cid = lax.axis_index("c")
    sid = lax.axis_index("s")

    _zero_rows(r0, CB)
    _zero_1d(zero1_v, CB)
    _zero_spmem(acc_sh, den_sh if with_cnt else None, r0, zero1_v, sid, ss)
    if with_cnt:
        one = jnp.ones((16,), jnp.float32)
        for q in range(Q):
            def ob(g, _, q=q):
                ones_v[q, pl.ds(g * 16, 16)] = one
                return 0

            lax.fori_loop(0, CB // 16, ob, 0)
    plsc.subcore_barrier()

    off = cid * NP

    def body(j, _):
        row0 = sid * (EPT // CB) + j * Q
        pltpu.sync_copy(src_hbm.at[pl.ds(row0, Q)], idxq)
        pltpu.sync_copy(dst_hbm.at[pl.ds(row0, Q)], dstq)
        for q in range(Q):
            def ib(g, _, q=q):
                sl = pl.ds(g * 16, 16)
                idxq[q, sl] = idxq[q, sl] + off
                return 0

            lax.fori_loop(0, CB // 16, ib, 0)
        rhs = []
        for q in range(Q):
            rhs.append(pltpu.async_copy(x_hbm.at[idxq.at[q]], rows[q], srs[q]))
        shs = []
        for q in range(Q):
            rhs[q].wait()
            shs.append(pltpu.async_copy(rows[q], acc_sh.at[dstq.at[q]], ss,
                                        add=True))
        if with_cnt:
            @pl.when(cid == 1)
            def _():
                for q in range(Q):
                    pltpu.sync_copy(ones_v.at[q], den_sh.at[dstq.at[q]],
                                    add=True)
        for h_ in shs:
            h_.wait()
        return 0

    lax.fori_loop(0, NBODY, body, 0)
    plsc.subcore_barrier()
    _writeback(acc_sh, acc_hbm, rows, sid, cid * NP, srs)

    if with_cnt:
        @pl.when(cid == 1)
        def _():
            _writeback_1d(den_sh, cnt_hbm, zero1_v, sid, ss)


def _sc_head_body(upn_idx_hbm, vid_idx_hbm, res_hbm, vidf_hbm,
                  upn_out, vid_out, idx_v, vidx_v, rows_v, vrow_v, sem):
    cid = lax.axis_index("c")
    sid = lax.axis_index("s")
    w = sid * 2 + cid

    pltpu.sync_copy(upn_idx_hbm.at[pl.ds(w * 3, 3)], idx_v)
    cps = []
    for k2 in range(3):
        cps.append(pltpu.async_copy(res_hbm.at[idx_v.at[k2]],
                                    rows_v.at[pl.ds(k2 * 128, 128)], sem))
    pltpu.sync_copy(vid_idx_hbm.at[pl.ds(w, 1)], vidx_v)
    cps.append(pltpu.async_copy(vidf_hbm.at[vidx_v.at[0]], vrow_v, sem))
    for cp in cps:
        cp.wait()
    pltpu.sync_copy(rows_v, upn_out.at[pl.ds(w * 384, 384)])
    pltpu.sync_copy(vrow_v, vid_out.at[pl.ds(w * 128, 128)])


def _sc_mesh():
    return plsc.VectorSubcoreMesh(core_axis_name="c", subcore_axis_name="s")


def _sc_gat(src2, dst2, a_s, a_d, ub16, h_flat):
    f32 = jnp.float32
    call = pl.kernel(
        _sc_gat_body,
        out_type=[
            jax.ShapeDtypeStruct((2 * NP, H), f32),
            jax.ShapeDtypeStruct((NP,), f32),
        ],
        mesh=_sc_mesh(),
        compiler_params=pltpu.CompilerParams(use_tc_tiling_on_sc=False),
        scratch_types=[
            pltpu.VMEM((Q, CB), jnp.int32),
            pltpu.VMEM((Q, CB), jnp.int32),
            pltpu.VMEM((Q, CB), jnp.int32),
            pltpu.VMEM((Q, CB), f32),
            pltpu.VMEM((Q, CB), f32),
            pltpu.VMEM((Q, CB), f32),
            pltpu.VMEM((CB, H), f32),
            pltpu.VMEM((CB, H), f32),
            pltpu.VMEM((CB, H), f32),
            pltpu.VMEM((CB, H), f32),
            pltpu.VMEM((16,), f32),
            pltpu.VMEM((CB,), f32),
            pltpu.VMEM_SHARED((NP, H), f32),
            pltpu.VMEM_SHARED((NP,), f32),
            pltpu.SemaphoreType.DMA,
            pltpu.SemaphoreType.DMA,
            pltpu.SemaphoreType.DMA,
            pltpu.SemaphoreType.DMA,
            pltpu.SemaphoreType.DMA,
            pltpu.SemaphoreType.DMA,
            pltpu.SemaphoreType.DMA,
            pltpu.SemaphoreType.DMA,
            pltpu.SemaphoreType.DMA,
        ],
    )
    return call(src2, dst2, a_s, a_d, ub16, h_flat)


def _sc_sage(src2, dst2, x_flat, with_cnt):
    f32 = jnp.float32
    out_type = [jax.ShapeDtypeStruct((2 * NP, H), f32)]
    if with_cnt:
        out_type.append(jax.ShapeDtypeStruct((NP,), f32))
    call = pl.kernel(
        functools.partial(_sc_sage_body, with_cnt=with_cnt),
        out_type=out_type,
        mesh=_sc_mesh(),
        compiler_params=pltpu.CompilerParams(use_tc_tiling_on_sc=False),
        scratch_types=[
            pltpu.VMEM((Q, CB), jnp.int32),
            pltpu.VMEM((Q, CB), jnp.int32),
            pltpu.VMEM((Q, CB), f32),
            pltpu.VMEM((CB, H), f32),
            pltpu.VMEM((CB, H), f32),
            pltpu.VMEM((CB, H), f32),
            pltpu.VMEM((CB, H), f32),
            pltpu.VMEM((CB,), f32),
            pltpu.VMEM_SHARED((NP, H), f32),
            pltpu.VMEM_SHARED((NP,), f32),
            pltpu.SemaphoreType.DMA,
            pltpu.SemaphoreType.DMA,
            pltpu.SemaphoreType.DMA,
            pltpu.SemaphoreType.DMA,
            pltpu.SemaphoreType.DMA,
        ],
    )
    return call(src2, dst2, x_flat)


def _sc_head(upn_idx2, vid_idx2, result, video_features):
    f32 = jnp.float32
    call = pl.kernel(
        _sc_head_body,
        out_type=[
            jax.ShapeDtypeStruct((3 * B, D), f32),
            jax.ShapeDtypeStruct((B, 128), f32),
        ],
        mesh=_sc_mesh(),
        compiler_params=pltpu.CompilerParams(use_tc_tiling_on_sc=False),
        scratch_types=[
            pltpu.VMEM((3, 128), jnp.int32),
            pltpu.VMEM((1, 128), jnp.int32),
            pltpu.VMEM((384, D), f32),
            pltpu.VMEM((128, 128), f32),
            pltpu.SemaphoreType.DMA,
        ],
    )
    return call(upn_idx2, vid_idx2, result, video_features)


def _pad_edges(ei):
    pad = E_PAD - E
    src = jnp.concatenate([ei[0], jnp.zeros((pad,), jnp.int32)])
    dst = jnp.concatenate([ei[1], jnp.full((pad,), TRASH, jnp.int32)])
    return src.reshape(E_PAD // CB, CB), dst.reshape(E_PAD // CB, CB)


def kernel(item, video_features, u_h_embedding, uh_edge_index, v_uh_edge_index,
           trans_W, trans_b, gat1_W, gat1_att, sage2_W, sage2_b,
           gat3_W, gat3_att, sage4_W, sage4_b, uv_W, uv_b, uh_W, uh_b):
    f32 = jnp.float32
    src_g, dst_g = _pad_edges(v_uh_edge_index)
    src_s, dst_s = _pad_edges(uh_edge_index)

    att1 = jnp.stack([gat1_att[:D], gat1_att[D:]])
    att3 = jnp.stack([gat3_att[:D], gat3_att[D:]])

    h1 = _gat1_prep(u_h_embedding, video_features, trans_W, trans_b, gat1_W)
    as1, ad1, mxs1, mxd1 = _attn_prep(h1, att1)
    ub1 = _leaky(mxs1[0, 0] + mxd1[0, 0], 0.2)
    acc1, den1 = _sc_gat(src_g, dst_g, as1.reshape(NP), ad1.reshape(NP),
                         jnp.full((16,), ub1, f32), h1.reshape(2 * NP, H))
    x1 = _div_leaky(acc1.reshape(2, NP, H), den1.reshape(NP, 1))

    s2, cnt = _sc_sage(src_s, dst_s, x1.reshape(2 * NP, H), with_cnt=True)
    cnt = cnt.reshape(NP, 1)

    h3, as3, ad3, mxs3, mxd3 = _sage_gat_prep(s2.reshape(2, NP, H), cnt,
                                              sage2_W, sage2_b, gat3_W, att3)
    ub3 = _leaky(mxs3[0, 0] + mxd3[0, 0], 0.2)
    acc3, den3 = _sc_gat(src_g, dst_g, as3.reshape(NP), ad3.reshape(NP),
                         jnp.full((16,), ub3, f32), h3.reshape(2 * NP, H))
    x3 = _div_leaky(acc3.reshape(2, NP, H), den3.reshape(NP, 1))

    (s4,) = _sc_sage(src_s, dst_s, x3.reshape(2 * NP, H), with_cnt=False)
    result = _sage_final(s4.reshape(2, NP, H), cnt, sage4_W, sage4_b)

    upn_idx = jnp.concatenate([item[:, 0], item[:, 2], item[:, 3]])
    upn_idx2 = upn_idx.reshape(3 * B // 128, 128)
    vid_idx2 = (item[:, 1] - N_UH).reshape(B // 128, 128)
    upn, vid = _sc_head(upn_idx2, vid_idx2, result, video_features)
    return _head(upn, vid, trans_W, trans_b, uv_W, uv_b, uh_W, uh_b)


# speedup vs baseline: 9.8282x; 1.0342x over previous
import functools

import jax
import jax.numpy as jnp
from jax import lax
from jax.experimental import pallas as pl
from jax.experimental.pallas import tpu as pltpu
from jax.experimental.pallas import tpu_sc as plsc

N_USER = 15000
N_UH = 25000
N_ALL = 50000
NP = 50176
D = 64
H = 32
B = 4096

E = 400000
EPT = 25600
E_PAD = EPT * 16
CB = 256
Q = 2
NBODY = EPT // (Q * CB)
TRASH = N_ALL

BLK = 512
BK = 7168
GK = NP // BK
AB = 5000

_TPR = NP // 16
_NFULL = _TPR // CB
_REM = _TPR % CB


def _leaky(x, s=0.01):
    return jnp.where(x >= 0, x, s * x)


def _gat1prep_body(uh_ref, vid_ref, tW_ref, tb_ref, W_ref, h_ref):
    r = pl.program_id(0)
    uh = uh_ref[...]
    nu = jnp.maximum(jnp.sqrt(jnp.sum(uh * uh, axis=1, keepdims=True)), 1e-12)
    xu = uh / nu
    v = _leaky(jnp.dot(vid_ref[...], tW_ref[...],
                       preferred_element_type=jnp.float32) + tb_ref[...])
    nv = jnp.maximum(jnp.sqrt(jnp.sum(v * v, axis=1, keepdims=True)), 1e-12)
    xv = v / nv
    x = jnp.where(r == 1, xv, xu)
    h = jnp.dot(x, W_ref[...], preferred_element_type=jnp.float32)
    c = pl.program_id(2)
    h_ref[0] = jnp.where(c == 0, h[:, :H], h[:, H:])


def _gat1_prep(u_h_embedding, video_features, trans_W, trans_b, W):
    full = lambda r, i, c: (0, 0)
    nb = N_UH // AB
    return pl.pallas_call(
        _gat1prep_body,
        grid=(2, nb, 2),
        in_specs=[
            pl.BlockSpec((AB, D), lambda r, i, c: (i, 0)),
            pl.BlockSpec((AB, 128), lambda r, i, c: (i, 0)),
            pl.BlockSpec((128, D), full),
            pl.BlockSpec((1, D), full),
            pl.BlockSpec((D, D), full),
        ],
        out_specs=pl.BlockSpec((1, AB, H),
                               lambda r, i, c: (c, r * (N_UH // AB) + i, 0)),
        out_shape=jax.ShapeDtypeStruct((2, NP, H), jnp.float32),
    )(u_h_embedding, video_features, trans_W, trans_b.reshape(1, D), W)


def _attn_body(ha_ref, hb_ref, att_ref, as_ref, ad_ref, mxs_ref, mxd_ref):
    i = pl.program_id(0)
    h = jnp.concatenate([ha_ref[0], hb_ref[0]], axis=1)
    a_s = jnp.dot(h, att_ref[...][0, :], preferred_element_type=jnp.float32)
    a_d = jnp.dot(h, att_ref[...][1, :], preferred_element_type=jnp.float32)
    as_ref[...] = a_s[:, None]
    ad_ref[...] = a_d[:, None]
    rid = i * BK + lax.broadcasted_iota(jnp.int32, (BK, 1), 0)
    mask = rid < N_ALL
    ms = jnp.max(jnp.where(mask, a_s[:, None], -1e30), axis=(0, 1),
                 keepdims=True)
    md = jnp.max(jnp.where(mask, a_d[:, None], -1e30), axis=(0, 1),
                 keepdims=True)
    neg = jnp.full((1, 1), -1e30, jnp.float32)
    prev_s = jnp.where(i == 0, neg, mxs_ref[...])
    prev_d = jnp.where(i == 0, neg, mxd_ref[...])
    mxs_ref[...] = jnp.maximum(prev_s, ms)
    mxd_ref[...] = jnp.maximum(prev_d, md)


def _attn_prep(h_packed, att2):
    full = lambda i: (0, 0)
    return pl.pallas_call(
        _attn_body,
        grid=(GK,),
        in_specs=[
            pl.BlockSpec((1, BK, H), lambda i: (0, i, 0)),
            pl.BlockSpec((1, BK, H), lambda i: (1, i, 0)),
            pl.BlockSpec((2, D), full),
        ],
        out_specs=[
            pl.BlockSpec((BK, 1), lambda i: (i, 0)),
            pl.BlockSpec((BK, 1), lambda i: (i, 0)),
            pl.BlockSpec((1, 1), full),
            pl.BlockSpec((1, 1), full),
        ],
        out_shape=[
            jax.ShapeDtypeStruct((NP, 1), jnp.float32),
            jax.ShapeDtypeStruct((NP, 1), jnp.float32),
            jax.ShapeDtypeStruct((1, 1), jnp.float32),
            jax.ShapeDtypeStruct((1, 1), jnp.float32),
        ],
    )(h_packed, h_packed, att2)


def _sagegatprep_body(xa_ref, xb_ref, cnt_ref, sW_ref, sb_ref, W_ref, att_ref,
                      h_ref, as_ref, ad_ref, mxs_ref, mxd_ref):
    i = pl.program_id(0)
    c = pl.program_id(1)
    s = jnp.concatenate([xa_ref[0], xb_ref[0]], axis=1)
    mean = s / jnp.maximum(cnt_ref[...], 1.0)
    x = _leaky(jnp.dot(mean, sW_ref[...],
                       preferred_element_type=jnp.float32) + sb_ref[...])
    h = jnp.dot(x, W_ref[...], preferred_element_type=jnp.float32)
    h_ref[0] = jnp.where(c == 0, h[:, :H], h[:, H:])

    @pl.when(c == 0)
    def _():
        a_s = jnp.dot(h, att_ref[...][0, :], preferred_element_type=jnp.float32)
        a_d = jnp.dot(h, att_ref[...][1, :], preferred_element_type=jnp.float32)
        as_ref[...] = a_s[:, None]
        ad_ref[...] = a_d[:, None]
        rid = i * BK + lax.broadcasted_iota(jnp.int32, (BK, 1), 0)
        mask = rid < N_ALL
        ms = jnp.max(jnp.where(mask, a_s[:, None], -1e30), axis=(0, 1),
                     keepdims=True)
        md = jnp.max(jnp.where(mask, a_d[:, None], -1e30), axis=(0, 1),
                     keepdims=True)
        neg = jnp.full((1, 1), -1e30, jnp.float32)
        prev_s = jnp.where(i == 0, neg, mxs_ref[...])
        prev_d = jnp.where(i == 0, neg, mxd_ref[...])
        mxs_ref[...] = jnp.maximum(prev_s, ms)
        mxd_ref[...] = jnp.maximum(prev_d, md)


def _sage_gat_prep(s_packed, cnt, sW, sb, W, att2):
    full = lambda i, c: (0, 0)
    return pl.pallas_call(
        _sagegatprep_body,
        grid=(GK, 2),
        in_specs=[
            pl.BlockSpec((1, BK, H), lambda i, c: (0, i, 0)),
            pl.BlockSpec((1, BK, H), lambda i, c: (1, i, 0)),
            pl.BlockSpec((BK, 1), lambda i, c: (i, 0)),
            pl.BlockSpec((D, D), full),
            pl.BlockSpec((1, D), full),
            pl.BlockSpec((D, D), full),
            pl.BlockSpec((2, D), full),
        ],
        out_specs=[
            pl.BlockSpec((1, BK, H), lambda i, c: (c, i, 0)),
            pl.BlockSpec((BK, 1), lambda i, c: (i, 0)),
            pl.BlockSpec((BK, 1), lambda i, c: (i, 0)),
            pl.BlockSpec((1, 1), full),
            pl.BlockSpec((1, 1), full),
        ],
        out_shape=[
            jax.ShapeDtypeStruct((2, NP, H), jnp.float32),
            jax.ShapeDtypeStruct((NP, 1), jnp.float32),
            jax.ShapeDtypeStruct((NP, 1), jnp.float32),
            jax.ShapeDtypeStruct((1, 1), jnp.float32),
            jax.ShapeDtypeStruct((1, 1), jnp.float32),
        ],
    )(s_packed, s_packed, cnt, sW, sb.reshape(1, D), W, att2)


def _divleaky_body(acc_ref, den_ref, x_ref):
    x_ref[0] = _leaky(acc_ref[0] / (den_ref[...] + 1e-16))


def _div_leaky(acc_packed, den):
    return pl.pallas_call(
        _divleaky_body,
        grid=(GK, 2),
        in_specs=[
            pl.BlockSpec((1, BK, H), lambda i, c: (c, i, 0)),
            pl.BlockSpec((BK, 1), lambda i, c: (i, 0)),
        ],
        out_specs=pl.BlockSpec((1, BK, H), lambda i, c: (c, i, 0)),
        out_shape=jax.ShapeDtypeStruct((2, NP, H), jnp.float32),
    )(acc_packed, den)


def _sage_final_body(sa_ref, sb2_ref, cnt_ref, W_ref, b_ref, out_ref):
    s = jnp.concatenate([sa_ref[0], sb2_ref[0]], axis=1)
    mean = s / jnp.maximum(cnt_ref[...], 1.0)
    out_ref[...] = _leaky(jnp.dot(mean, W_ref[...],
                                  preferred_element_type=jnp.float32) + b_ref[...])


def _sage_final(s_packed, cnt, W, b):
    fb = BK // 2
    return pl.pallas_call(
        _sage_final_body,
        grid=(-(-N_UH // fb),),
        in_specs=[
            pl.BlockSpec((1, fb, H), lambda i: (0, i, 0)),
            pl.BlockSpec((1, fb, H), lambda i: (1, i, 0)),
            pl.BlockSpec((fb, 1), lambda i: (i, 0)),
            pl.BlockSpec((D, D), lambda i: (0, 0)),
            pl.BlockSpec((1, D), lambda i: (0, 0)),
        ],
        out_specs=pl.BlockSpec((fb, D), lambda i: (i, 0)),
        out_shape=jax.ShapeDtypeStruct((N_UH, D), jnp.float32),
    )(s_packed, s_packed, cnt, W, b.reshape(1, D))


def _head_body(user_ref, pos_ref, neg_ref, vid_ref, tW_ref, tb_ref,
               uvW_ref, uvb_ref, uhW_ref, uhb_ref, ps_ref, ns_ref):
    vid_t = _leaky(jnp.dot(vid_ref[...], tW_ref[...],
                           preferred_element_type=jnp.float32) + tb_ref[...])
    user = user_ref[...]
    usv = _leaky(jnp.dot(jnp.concatenate([vid_t, user], axis=1), uvW_ref[...],
                         preferred_element_type=jnp.float32) + uvb_ref[...])
    usph = _leaky(jnp.dot(jnp.concatenate([pos_ref[...], user], axis=1),
                          uhW_ref[...],
                          preferred_element_type=jnp.float32) + uhb_ref[...])
    usnh = _leaky(jnp.dot(jnp.concatenate([neg_ref[...], user], axis=1),
                          uhW_ref[...],
                          preferred_element_type=jnp.float32) + uhb_ref[...])
    ps_ref[...] = jnp.sum(usv * usph, axis=1, keepdims=True)
    ns_ref[...] = jnp.sum(usv * usnh, axis=1, keepdims=True)


def _head(upn, vid, tW, tb, uvW, uvb, uhW, uhb):
    row = lambda i: (i, 0)
    full = lambda i: (0, 0)
    ps, ns = pl.pallas_call(
        _head_body,
        grid=(B // BLK,),
        in_specs=[
            pl.BlockSpec((BLK, D), lambda i: (i, 0)),
            pl.BlockSpec((BLK, D), lambda i: (8 + i, 0)),
            pl.BlockSpec((BLK, D), lambda i: (16 + i, 0)),
            pl.BlockSpec((BLK, 128), row),
            pl.BlockSpec((128, D), full),
            pl.BlockSpec((1, D), full),
            pl.BlockSpec((128, D), full),
            pl.BlockSpec((1, D), full),
            pl.BlockSpec((128, D), full),
            pl.BlockSpec((1, D), full),
        ],
        out_specs=[pl.BlockSpec((BLK, 1), row), pl.BlockSpec((BLK, 1), row)],
        out_shape=[jax.ShapeDtypeStruct((B, 1), jnp.float32),
                   jax.ShapeDtypeStruct((B, 1), jnp.float32)],
    )(upn, upn, upn, vid, tW, tb.reshape(1, D), uvW, uvb.reshape(1, D),
      uhW, uhb.reshape(1, D))
    return ps.reshape(B), ns.reshape(B)


def _zero_rows(rows_v, n):
    zero = jnp.zeros((16,), jnp.float32)

    def zb(i, _):
        rows_v[i, pl.ds(0, 16)] = zero
        rows_v[i, pl.ds(16, 16)] = zero
        return 0

    lax.fori_loop(0, n, zb, 0)


def _zero_1d(ref, n):
    zero = jnp.zeros((16,), jnp.float32)

    def zb(i, _):
        ref[pl.ds(i * 16, 16)] = zero
        return 0

    lax.fori_loop(0, n // 16, zb, 0)


def _zero_spmem(acc_sh, den_sh, zrow_v, zero1_v, sid, sem):
    base = sid * _TPR
    hs = []
    for t in range(_NFULL):
        hs.append(pltpu.async_copy(zrow_v, acc_sh.at[pl.ds(base + t * CB, CB)],
                                   sem))
    hs.append(pltpu.async_copy(zrow_v.at[pl.ds(0, _REM)],
                               acc_sh.at[pl.ds(base + _NFULL * CB, _REM)], sem))
    if den_sh is not None:
        for t in range(_NFULL):
            hs.append(pltpu.async_copy(zero1_v,
                                       den_sh.at[pl.ds(base + t * CB, CB)], sem))
        hs.append(pltpu.async_copy(zero1_v.at[pl.ds(0, _REM)],
                                   den_sh.at[pl.ds(base + _NFULL * CB, _REM)],
                                   sem))
    for h_ in hs:
        h_.wait()


def _writeback(acc_sh, acc_hbm, rows, sid, out_base, sems):
    base = sid * _TPR
    hs = [None] * Q
    for t in range(_NFULL):
        b = t % Q
        if hs[b] is not None:
            hs[b].wait()
        pltpu.sync_copy(acc_sh.at[pl.ds(base + t * CB, CB)], rows[b])
        hs[b] = pltpu.async_copy(rows[b],
                                 acc_hbm.at[pl.ds(out_base + base + t * CB, CB)],
                                 sems[b])
    b = _NFULL % Q
    if hs[b] is not None:
        hs[b].wait()
        hs[b] = None
    pltpu.sync_copy(acc_sh.at[pl.ds(base + _NFULL * CB, _REM)],
                    rows[b].at[pl.ds(0, _REM)])
    hs[b] = pltpu.async_copy(rows[b].at[pl.ds(0, _REM)],
                             acc_hbm.at[pl.ds(out_base + base + _NFULL * CB,
                                              _REM)], sems[b])
    for h_ in hs:
        if h_ is not None:
            h_.wait()


def _writeback_1d(den_sh, den_hbm, zero1_v, sid, sem):
    base = sid * _TPR
    for t in range(_NFULL):
        pltpu.sync_copy(den_sh.at[pl.ds(base + t * CB, CB)], zero1_v)
        pltpu.sync_copy(zero1_v, den_hbm.at[pl.ds(base + t * CB, CB)])
    pltpu.sync_copy(den_sh.at[pl.ds(base + _NFULL * CB, _REM)],
                    zero1_v.at[pl.ds(0, _REM)])
    pltpu.sync_copy(zero1_v.at[pl.ds(0, _REM)],
                    den_hbm.at[pl.ds(base + _NFULL * CB, _REM)])


def _sc_gat_body(src_hbm, dst_hbm, as_hbm, ad_hbm, ub_hbm, h_hbm,
                 acc_hbm, den_hbm,
                 idxq, idxo, dstq, asg, adg, wq, r0, r1, ub_v, zero1_v,
                 acc_sh, den_sh,
                 sa0, sa1, sr0, sr1, ss):
    rows = (r0, r1)
    sas = (sa0, sa1)
    srs = (sr0, sr1)
    cid = lax.axis_index("c")
    sid = lax.axis_index("s")

    _zero_rows(r0, CB)
    _zero_1d(zero1_v, CB)
    _zero_spmem(acc_sh, den_sh, r0, zero1_v, sid, ss)
    pltpu.sync_copy(ub_hbm, ub_v)
    plsc.subcore_barrier()

    ubv = ub_v[...]
    off = cid * NP

    def body(j, _):
        row0 = sid * (EPT // CB) + j * Q
        pltpu.sync_copy(src_hbm.at[pl.ds(row0, Q)], idxq)
        pltpu.sync_copy(dst_hbm.at[pl.ds(row0, Q)], dstq)
        ghs = []
        for q in range(Q):
            ghs.append((pltpu.async_copy(as_hbm.at[idxq.at[q]], asg.at[q],
                                         sas[q]),
                        pltpu.async_copy(ad_hbm.at[dstq.at[q]], adg.at[q],
                                         sas[q])))
        for q in range(Q):
            def ib(g, _, q=q):
                sl = pl.ds(g * 16, 16)
                idxo[q, sl] = idxq[q, sl] + off
                return 0

            lax.fori_loop(0, CB // 16, ib, 0)
        rhs = []
        for q in range(Q):
            rhs.append(pltpu.async_copy(h_hbm.at[idxo.at[q]], rows[q], srs[q]))
        shs = []
        for q in range(Q):
            ghs[q][0].wait()
            ghs[q][1].wait()

            def wb(g, _, q=q):
                sl = pl.ds(g * 16, 16)
                a = asg[q, sl] + adg[q, sl]
                alpha = jnp.where(a >= 0, a, 0.2 * a)
                wq[q, sl] = jnp.exp(alpha - ubv)
                return 0

            lax.fori_loop(0, CB // 16, wb, 0)
            rhs[q].wait()

            def sb(g, _, q=q):
                wv = wq[q, pl.ds(g * 16, 16)]
                for lane in range(16):
                    ws = wv[lane]
                    rr = g * 16 + lane
                    rows[q][rr, pl.ds(0, 16)] = rows[q][rr, pl.ds(0, 16)] * ws
                    rows[q][rr, pl.ds(16, 16)] = rows[q][rr, pl.ds(16, 16)] * ws
                return 0

            lax.fori_loop(0, CB // 16, sb, 0)
            shs.append(pltpu.async_copy(rows[q], acc_sh.at[dstq.at[q]], ss,
                                        add=True))

        @pl.when(cid == 1)
        def _():
            for q in range(Q):
                pltpu.sync_copy(wq.at[q], den_sh.at[dstq.at[q]], add=True)

        for h_ in shs:
            h_.wait()
        return 0

    lax.fori_loop(0, NBODY, body, 0)
    plsc.subcore_barrier()
    _writeback(acc_sh, acc_hbm, rows, sid, cid * NP, srs)

    @pl.when(cid == 1)
    def _():
        _writeback_1d(den_sh, den_hbm, zero1_v, sid, ss)


def _sc_sage_body(src_hbm, dst_hbm, x_hbm, *args, with_cnt):
    if with_cnt:
        (acc_hbm, cnt_hbm, idxq, dstq, ones_v, r0, r1, zero1_v,
         acc_sh, den_sh, sr0, sr1, ss) = args
    else:
        (acc_hbm, idxq, dstq, ones_v, r0, r1, zero1_v,
         acc_sh, den_sh, sr0, sr1, ss) = args
    rows = (r0, r1)
    srs = (sr0, sr1)
    cid = lax.axis_index("c")
    sid = lax.axis_index("s")

    _zero_rows(r0, CB)
    _zero_1d(zero1_v, CB)
    _zero_spmem(acc_sh, den_sh if with_cnt else None, r0, zero1_v, sid, ss)
    if with_cnt:
        one = jnp.ones((16,), jnp.float32)
        for q in range(Q):
            def ob(g, _, q=q):
                ones_v[q, pl.ds(g * 16, 16)] = one
                return 0

            lax.fori_loop(0, CB // 16, ob, 0)
    plsc.subcore_barrier()

    off = cid * NP

    def body(j, _):
        row0 = sid * (EPT // CB) + j * Q
        pltpu.sync_copy(src_hbm.at[pl.ds(row0, Q)], idxq)
        pltpu.sync_copy(dst_hbm.at[pl.ds(row0, Q)], dstq)
        for q in range(Q):
            def ib(g, _, q=q):
                sl = pl.ds(g * 16, 16)
                idxq[q, sl] = idxq[q, sl] + off
                return 0

            lax.fori_loop(0, CB // 16, ib, 0)
        rhs = []
        for q in range(Q):
            rhs.append(pltpu.async_copy(x_hbm.at[idxq.at[q]], rows[q], srs[q]))
        shs = []
        for q in range(Q):
            rhs[q].wait()
            shs.append(pltpu.async_copy(rows[q], acc_sh.at[dstq.at[q]], ss,
                                        add=True))
        if with_cnt:
            @pl.when(cid == 1)
            def _():
                for q in range(Q):
                    pltpu.sync_copy(ones_v.at[q], den_sh.at[dstq.at[q]],
                                    add=True)
        for h_ in shs:
            h_.wait()
        return 0

    lax.fori_loop(0, NBODY, body, 0)
    plsc.subcore_barrier()
    _writeback(acc_sh, acc_hbm, rows, sid, cid * NP, srs)

    if with_cnt:
        @pl.when(cid == 1)
        def _():
            _writeback_1d(den_sh, cnt_hbm, zero1_v, sid, ss)


def _sc_head_body(upn_idx_hbm, vid_idx_hbm, res_hbm, vidf_hbm,
                  upn_out, vid_out, idx_v, vidx_v, rows_v, vrow_v, sem):
    cid = lax.axis_index("c")
    sid = lax.axis_index("s")
    w = sid * 2 + cid

    pltpu.sync_copy(upn_idx_hbm.at[pl.ds(w * 3, 3)], idx_v)
    cps = []
    for k2 in range(3):
        cps.append(pltpu.async_copy(res_hbm.at[idx_v.at[k2]],
                                    rows_v.at[pl.ds(k2 * 128, 128)], sem))
    pltpu.sync_copy(vid_idx_hbm.at[pl.ds(w, 1)], vidx_v)
    cps.append(pltpu.async_copy(vidf_hbm.at[vidx_v.at[0]], vrow_v, sem))
    for cp in cps:
        cp.wait()
    pltpu.sync_copy(rows_v, upn_out.at[pl.ds(w * 384, 384)])
    pltpu.sync_copy(vrow_v, vid_out.at[pl.ds(w * 128, 128)])


def _sc_mesh():
    return plsc.VectorSubcoreMesh(core_axis_name="c", subcore_axis_name="s")


def _sc_gat(src2, dst2, a_s, a_d, ub16, h_flat):
    f32 = jnp.float32
    call = pl.kernel(
        _sc_gat_body,
        out_type=[
            jax.ShapeDtypeStruct((2 * NP, H), f32),
            jax.ShapeDtypeStruct((NP,), f32),
        ],
        mesh=_sc_mesh(),
        compiler_params=pltpu.CompilerParams(use_tc_tiling_on_sc=False),
        scratch_types=[
            pltpu.VMEM((Q, CB), jnp.int32),
            pltpu.VMEM((Q, CB), jnp.int32),
            pltpu.VMEM((Q, CB), jnp.int32),
            pltpu.VMEM((Q, CB), f32),
            pltpu.VMEM((Q, CB), f32),
            pltpu.VMEM((Q, CB), f32),
            pltpu.VMEM((CB, H), f32),
            pltpu.VMEM((CB, H), f32),
            pltpu.VMEM((16,), f32),
            pltpu.VMEM((CB,), f32),
            pltpu.VMEM_SHARED((NP, H), f32),
            pltpu.VMEM_SHARED((NP,), f32),
            pltpu.SemaphoreType.DMA,
            pltpu.SemaphoreType.DMA,
            pltpu.SemaphoreType.DMA,
            pltpu.SemaphoreType.DMA,
            pltpu.SemaphoreType.DMA,
        ],
    )
    return call(src2, dst2, a_s, a_d, ub16, h_flat)


def _sc_sage(src2, dst2, x_flat, with_cnt):
    f32 = jnp.float32
    out_type = [jax.ShapeDtypeStruct((2 * NP, H), f32)]
    if with_cnt:
        out_type.append(jax.ShapeDtypeStruct((NP,), f32))
    call = pl.kernel(
        functools.partial(_sc_sage_body, with_cnt=with_cnt),
        out_type=out_type,
        mesh=_sc_mesh(),
        compiler_params=pltpu.CompilerParams(use_tc_tiling_on_sc=False),
        scratch_types=[
            pltpu.VMEM((Q, CB), jnp.int32),
            pltpu.VMEM((Q, CB), jnp.int32),
            pltpu.VMEM((Q, CB), f32),
            pltpu.VMEM((CB, H), f32),
            pltpu.VMEM((CB, H), f32),
            pltpu.VMEM((CB,), f32),
            pltpu.VMEM_SHARED((NP, H), f32),
            pltpu.VMEM_SHARED((NP,), f32),
            pltpu.SemaphoreType.DMA,
            pltpu.SemaphoreType.DMA,
            pltpu.SemaphoreType.DMA,
        ],
    )
    return call(src2, dst2, x_flat)


def _sc_head(upn_idx2, vid_idx2, result, video_features):
    f32 = jnp.float32
    call = pl.kernel(
        _sc_head_body,
        out_type=[
            jax.ShapeDtypeStruct((3 * B, D), f32),
            jax.ShapeDtypeStruct((B, 128), f32),
        ],
        mesh=_sc_mesh(),
        compiler_params=pltpu.CompilerParams(use_tc_tiling_on_sc=False),
        scratch_types=[
            pltpu.VMEM((3, 128), jnp.int32),
            pltpu.VMEM((1, 128), jnp.int32),
            pltpu.VMEM((384, D), f32),
            pltpu.VMEM((128, 128), f32),
            pltpu.SemaphoreType.DMA,
        ],
    )
    return call(upn_idx2, vid_idx2, result, video_features)


def _pad_edges(ei):
    pad = E_PAD - E
    src = jnp.concatenate([ei[0], jnp.zeros((pad,), jnp.int32)])
    dst = jnp.concatenate([ei[1], jnp.full((pad,), TRASH, jnp.int32)])
    return src.reshape(E_PAD // CB, CB), dst.reshape(E_PAD // CB, CB)


def kernel(item, video_features, u_h_embedding, uh_edge_index, v_uh_edge_index,
           trans_W, trans_b, gat1_W, gat1_att, sage2_W, sage2_b,
           gat3_W, gat3_att, sage4_W, sage4_b, uv_W, uv_b, uh_W, uh_b):
    f32 = jnp.float32
    src_g, dst_g = _pad_edges(v_uh_edge_index)
    src_s, dst_s = _pad_edges(uh_edge_index)

    att1 = jnp.stack([gat1_att[:D], gat1_att[D:]])
    att3 = jnp.stack([gat3_att[:D], gat3_att[D:]])

    h1 = _gat1_prep(u_h_embedding, video_features, trans_W, trans_b, gat1_W)
    as1, ad1, mxs1, mxd1 = _attn_prep(h1, att1)
    ub1 = _leaky(mxs1[0, 0] + mxd1[0, 0], 0.2)
    acc1, den1 = _sc_gat(src_g, dst_g, as1.reshape(NP), ad1.reshape(NP),
                         jnp.full((16,), ub1, f32), h1.reshape(2 * NP, H))
    x1 = _div_leaky(acc1.reshape(2, NP, H), den1.reshape(NP, 1))

    s2, cnt = _sc_sage(src_s, dst_s, x1.reshape(2 * NP, H), with_cnt=True)
    cnt = cnt.reshape(NP, 1)

    h3, as3, ad3, mxs3, mxd3 = _sage_gat_prep(s2.reshape(2, NP, H), cnt,
                                              sage2_W, sage2_b, gat3_W, att3)
    ub3 = _leaky(mxs3[0, 0] + mxd3[0, 0], 0.2)
    acc3, den3 = _sc_gat(src_g, dst_g, as3.reshape(NP), ad3.reshape(NP),
                         jnp.full((16,), ub3, f32), h3.reshape(2 * NP, H))
    x3 = _div_leaky(acc3.reshape(2, NP, H), den3.reshape(NP, 1))

    (s4,) = _sc_sage(src_s, dst_s, x3.reshape(2 * NP, H), with_cnt=False)
    result = _sage_final(s4.reshape(2, NP, H), cnt, sage4_W, sage4_b)

    upn_idx = jnp.concatenate([item[:, 0], item[:, 2], item[:, 3]])
    upn_idx2 = upn_idx.reshape(3 * B // 128, 128)
    vid_idx2 = (item[:, 1] - N_UH).reshape(B // 128, 128)
    upn, vid = _sc_head(upn_idx2, vid_idx2, result, video_features)
    return _head(upn, vid, trans_W, trans_b, uv_W, uv_b, uh_W, uh_b)


# speedup vs baseline: 10.7891x; 1.0978x over previous
import functools

import jax
import jax.numpy as jnp
from jax import lax
from jax.experimental import pallas as pl
from jax.experimental.pallas import tpu as pltpu
from jax.experimental.pallas import tpu_sc as plsc

N_USER = 15000
N_UH = 25000
N_ALL = 50000
NP = 50176
D = 64
H = 32
B = 4096

E = 400000
EPT = 25600
E_PAD = EPT * 16
CB = 256
Q = 2
NBODY = EPT // (Q * CB)
TRASH = N_ALL

BLK = 512
BK = 7168
GK = NP // BK
AB = 5000

_TPR = NP // 16
_NFULL = _TPR // CB
_REM = _TPR % CB


def _leaky(x, s=0.01):
    return jnp.where(x >= 0, x, s * x)


def _gat1prep_body(uh_ref, vid_ref, tW_ref, tb_ref, W_ref, h_ref):
    r = pl.program_id(0)
    uh = uh_ref[...]
    nu = jnp.maximum(jnp.sqrt(jnp.sum(uh * uh, axis=1, keepdims=True)), 1e-12)
    xu = uh / nu
    v = _leaky(jnp.dot(vid_ref[...], tW_ref[...],
                       preferred_element_type=jnp.float32) + tb_ref[...])
    nv = jnp.maximum(jnp.sqrt(jnp.sum(v * v, axis=1, keepdims=True)), 1e-12)
    xv = v / nv
    x = jnp.where(r == 1, xv, xu)
    h = jnp.dot(x, W_ref[...], preferred_element_type=jnp.float32)
    c = pl.program_id(2)
    h_ref[0] = jnp.where(c == 0, h[:, :H], h[:, H:])


def _gat1_prep(u_h_embedding, video_features, trans_W, trans_b, W):
    full = lambda r, i, c: (0, 0)
    nb = N_UH // AB
    return pl.pallas_call(
        _gat1prep_body,
        grid=(2, nb, 2),
        in_specs=[
            pl.BlockSpec((AB, D), lambda r, i, c: (i, 0)),
            pl.BlockSpec((AB, 128), lambda r, i, c: (i, 0)),
            pl.BlockSpec((128, D), full),
            pl.BlockSpec((1, D), full),
            pl.BlockSpec((D, D), full),
        ],
        out_specs=pl.BlockSpec((1, AB, H),
                               lambda r, i, c: (c, r * (N_UH // AB) + i, 0)),
        out_shape=jax.ShapeDtypeStruct((2, NP, H), jnp.float32),
    )(u_h_embedding, video_features, trans_W, trans_b.reshape(1, D), W)


def _attn_body(ha_ref, hb_ref, att_ref, as_ref, ad_ref, mxs_ref, mxd_ref):
    i = pl.program_id(0)
    h = jnp.concatenate([ha_ref[0], hb_ref[0]], axis=1)
    a_s = jnp.dot(h, att_ref[...][0, :], preferred_element_type=jnp.float32)
    a_d = jnp.dot(h, att_ref[...][1, :], preferred_element_type=jnp.float32)
    as_ref[...] = a_s[:, None]
    ad_ref[...] = a_d[:, None]
    rid = i * BK + lax.broadcasted_iota(jnp.int32, (BK, 1), 0)
    mask = rid < N_ALL
    ms = jnp.max(jnp.where(mask, a_s[:, None], -1e30), axis=(0, 1),
                 keepdims=True)
    md = jnp.max(jnp.where(mask, a_d[:, None], -1e30), axis=(0, 1),
                 keepdims=True)
    neg = jnp.full((1, 1), -1e30, jnp.float32)
    prev_s = jnp.where(i == 0, neg, mxs_ref[...])
    prev_d = jnp.where(i == 0, neg, mxd_ref[...])
    mxs_ref[...] = jnp.maximum(prev_s, ms)
    mxd_ref[...] = jnp.maximum(prev_d, md)


def _attn_prep(h_packed, att2):
    full = lambda i: (0, 0)
    return pl.pallas_call(
        _attn_body,
        grid=(GK,),
        in_specs=[
            pl.BlockSpec((1, BK, H), lambda i: (0, i, 0)),
            pl.BlockSpec((1, BK, H), lambda i: (1, i, 0)),
            pl.BlockSpec((2, D), full),
        ],
        out_specs=[
            pl.BlockSpec((BK, 1), lambda i: (i, 0)),
            pl.BlockSpec((BK, 1), lambda i: (i, 0)),
            pl.BlockSpec((1, 1), full),
            pl.BlockSpec((1, 1), full),
        ],
        out_shape=[
            jax.ShapeDtypeStruct((NP, 1), jnp.float32),
            jax.ShapeDtypeStruct((NP, 1), jnp.float32),
            jax.ShapeDtypeStruct((1, 1), jnp.float32),
            jax.ShapeDtypeStruct((1, 1), jnp.float32),
        ],
    )(h_packed, h_packed, att2)


def _sagegatprep_body(xa_ref, xb_ref, cnt_ref, sW_ref, sb_ref, W_ref, att_ref,
                      h_ref, as_ref, ad_ref, mxs_ref, mxd_ref):
    i = pl.program_id(0)
    c = pl.program_id(1)
    s = jnp.concatenate([xa_ref[0], xb_ref[0]], axis=1)
    mean = s / jnp.maximum(cnt_ref[...], 1.0)
    x = _leaky(jnp.dot(mean, sW_ref[...],
                       preferred_element_type=jnp.float32) + sb_ref[...])
    h = jnp.dot(x, W_ref[...], preferred_element_type=jnp.float32)
    h_ref[0] = jnp.where(c == 0, h[:, :H], h[:, H:])

    @pl.when(c == 0)
    def _():
        a_s = jnp.dot(h, att_ref[...][0, :], preferred_element_type=jnp.float32)
        a_d = jnp.dot(h, att_ref[...][1, :], preferred_element_type=jnp.float32)
        as_ref[...] = a_s[:, None]
        ad_ref[...] = a_d[:, None]
        rid = i * BK + lax.broadcasted_iota(jnp.int32, (BK, 1), 0)
        mask = rid < N_ALL
        ms = jnp.max(jnp.where(mask, a_s[:, None], -1e30), axis=(0, 1),
                     keepdims=True)
        md = jnp.max(jnp.where(mask, a_d[:, None], -1e30), axis=(0, 1),
                     keepdims=True)
        neg = jnp.full((1, 1), -1e30, jnp.float32)
        prev_s = jnp.where(i == 0, neg, mxs_ref[...])
        prev_d = jnp.where(i == 0, neg, mxd_ref[...])
        mxs_ref[...] = jnp.maximum(prev_s, ms)
        mxd_ref[...] = jnp.maximum(prev_d, md)


def _sage_gat_prep(s_packed, cnt, sW, sb, W, att2):
    full = lambda i, c: (0, 0)
    return pl.pallas_call(
        _sagegatprep_body,
        grid=(GK, 2),
        in_specs=[
            pl.BlockSpec((1, BK, H), lambda i, c: (0, i, 0)),
            pl.BlockSpec((1, BK, H), lambda i, c: (1, i, 0)),
            pl.BlockSpec((BK, 1), lambda i, c: (i, 0)),
            pl.BlockSpec((D, D), full),
            pl.BlockSpec((1, D), full),
            pl.BlockSpec((D, D), full),
            pl.BlockSpec((2, D), full),
        ],
        out_specs=[
            pl.BlockSpec((1, BK, H), lambda i, c: (c, i, 0)),
            pl.BlockSpec((BK, 1), lambda i, c: (i, 0)),
            pl.BlockSpec((BK, 1), lambda i, c: (i, 0)),
            pl.BlockSpec((1, 1), full),
            pl.BlockSpec((1, 1), full),
        ],
        out_shape=[
            jax.ShapeDtypeStruct((2, NP, H), jnp.float32),
            jax.ShapeDtypeStruct((NP, 1), jnp.float32),
            jax.ShapeDtypeStruct((NP, 1), jnp.float32),
            jax.ShapeDtypeStruct((1, 1), jnp.float32),
            jax.ShapeDtypeStruct((1, 1), jnp.float32),
        ],
    )(s_packed, s_packed, cnt, sW, sb.reshape(1, D), W, att2)


def _divleaky_body(acc_ref, den_ref, x_ref):
    x_ref[0] = _leaky(acc_ref[0] / (den_ref[...] + 1e-16))


def _div_leaky(acc_packed, den):
    return pl.pallas_call(
        _divleaky_body,
        grid=(GK, 2),
        in_specs=[
            pl.BlockSpec((1, BK, H), lambda i, c: (c, i, 0)),
            pl.BlockSpec((BK, 1), lambda i, c: (i, 0)),
        ],
        out_specs=pl.BlockSpec((1, BK, H), lambda i, c: (c, i, 0)),
        out_shape=jax.ShapeDtypeStruct((2, NP, H), jnp.float32),
    )(acc_packed, den)


def _sage_final_body(sa_ref, sb2_ref, cnt_ref, W_ref, b_ref, out_ref):
    s = jnp.concatenate([sa_ref[0], sb2_ref[0]], axis=1)
    mean = s / jnp.maximum(cnt_ref[...], 1.0)
    out_ref[...] = _leaky(jnp.dot(mean, W_ref[...],
                                  preferred_element_type=jnp.float32) + b_ref[...])


def _sage_final(s_packed, cnt, W, b):
    fb = BK // 2
    return pl.pallas_call(
        _sage_final_body,
        grid=(-(-N_UH // fb),),
        in_specs=[
            pl.BlockSpec((1, fb, H), lambda i: (0, i, 0)),
            pl.BlockSpec((1, fb, H), lambda i: (1, i, 0)),
            pl.BlockSpec((fb, 1), lambda i: (i, 0)),
            pl.BlockSpec((D, D), lambda i: (0, 0)),
            pl.BlockSpec((1, D), lambda i: (0, 0)),
        ],
        out_specs=pl.BlockSpec((fb, D), lambda i: (i, 0)),
        out_shape=jax.ShapeDtypeStruct((N_UH, D), jnp.float32),
    )(s_packed, s_packed, cnt, W, b.reshape(1, D))


def _head_body(user_ref, pos_ref, neg_ref, vid_ref, tW_ref, tb_ref,
               uvW_ref, uvb_ref, uhW_ref, uhb_ref, ps_ref, ns_ref):
    vid_t = _leaky(jnp.dot(vid_ref[...], tW_ref[...],
                           preferred_element_type=jnp.float32) + tb_ref[...])
    user = user_ref[...]
    usv = _leaky(jnp.dot(jnp.concatenate([vid_t, user], axis=1), uvW_ref[...],
                         preferred_element_type=jnp.float32) + uvb_ref[...])
    usph = _leaky(jnp.dot(jnp.concatenate([pos_ref[...], user], axis=1),
                          uhW_ref[...],
                          preferred_element_type=jnp.float32) + uhb_ref[...])
    usnh = _leaky(jnp.dot(jnp.concatenate([neg_ref[...], user], axis=1),
                          uhW_ref[...],
                          preferred_element_type=jnp.float32) + uhb_ref[...])
    ps_ref[...] = jnp.sum(usv * usph, axis=1, keepdims=True)
    ns_ref[...] = jnp.sum(usv * usnh, axis=1, keepdims=True)


def _head(upn, vid, tW, tb, uvW, uvb, uhW, uhb):
    row = lambda i: (i, 0)
    full = lambda i: (0, 0)
    ps, ns = pl.pallas_call(
        _head_body,
        grid=(B // BLK,),
        in_specs=[
            pl.BlockSpec((BLK, D), lambda i: (i, 0)),
            pl.BlockSpec((BLK, D), lambda i: (8 + i, 0)),
            pl.BlockSpec((BLK, D), lambda i: (16 + i, 0)),
            pl.BlockSpec((BLK, 128), row),
            pl.BlockSpec((128, D), full),
            pl.BlockSpec((1, D), full),
            pl.BlockSpec((128, D), full),
            pl.BlockSpec((1, D), full),
            pl.BlockSpec((128, D), full),
            pl.BlockSpec((1, D), full),
        ],
        out_specs=[pl.BlockSpec((BLK, 1), row), pl.BlockSpec((BLK, 1), row)],
        out_shape=[jax.ShapeDtypeStruct((B, 1), jnp.float32),
                   jax.ShapeDtypeStruct((B, 1), jnp.float32)],
    )(upn, upn, upn, vid, tW, tb.reshape(1, D), uvW, uvb.reshape(1, D),
      uhW, uhb.reshape(1, D))
    return ps.reshape(B), ns.reshape(B)


def _zero_rows(rows_v, n):
    zero = jnp.zeros((16,), jnp.float32)

    def zb(i, _):
        rows_v[i, pl.ds(0, 16)] = zero
        rows_v[i, pl.ds(16, 16)] = zero
        return 0

    lax.fori_loop(0, n, zb, 0)


def _zero_1d(ref, n):
    zero = jnp.zeros((16,), jnp.float32)

    def zb(i, _):
        ref[pl.ds(i * 16, 16)] = zero
        return 0

    lax.fori_loop(0, n // 16, zb, 0)


def _zero_spmem(acc_sh, den_sh, zrow_v, zero1_v, sid, sem):
    base = sid * _TPR
    hs = []
    for t in range(_NFULL):
        hs.append(pltpu.async_copy(zrow_v, acc_sh.at[pl.ds(base + t * CB, CB)],
                                   sem))
    hs.append(pltpu.async_copy(zrow_v.at[pl.ds(0, _REM)],
                               acc_sh.at[pl.ds(base + _NFULL * CB, _REM)], sem))
    if den_sh is not None:
        for t in range(_NFULL):
            hs.append(pltpu.async_copy(zero1_v,
                                       den_sh.at[pl.ds(base + t * CB, CB)], sem))
        hs.append(pltpu.async_copy(zero1_v.at[pl.ds(0, _REM)],
                                   den_sh.at[pl.ds(base + _NFULL * CB, _REM)],
                                   sem))
    for h_ in hs:
        h_.wait()


def _writeback(acc_sh, acc_hbm, rows, sid, out_base, sems):
    base = sid * _TPR
    hs = [None] * Q
    for t in range(_NFULL):
        b = t % Q
        if hs[b] is not None:
            hs[b].wait()
        pltpu.sync_copy(acc_sh.at[pl.ds(base + t * CB, CB)], rows[b])
        hs[b] = pltpu.async_copy(rows[b],
                                 acc_hbm.at[pl.ds(out_base + base + t * CB, CB)],
                                 sems[b])
    b = _NFULL % Q
    if hs[b] is not None:
        hs[b].wait()
        hs[b] = None
    pltpu.sync_copy(acc_sh.at[pl.ds(base + _NFULL * CB, _REM)],
                    rows[b].at[pl.ds(0, _REM)])
    hs[b] = pltpu.async_copy(rows[b].at[pl.ds(0, _REM)],
                             acc_hbm.at[pl.ds(out_base + base + _NFULL * CB,
                                              _REM)], sems[b])
    for h_ in hs:
        if h_ is not None:
            h_.wait()


def _writeback_1d(den_sh, den_hbm, zero1_v, sid, sem):
    base = sid * _TPR
    for t in range(_NFULL):
        pltpu.sync_copy(den_sh.at[pl.ds(base + t * CB, CB)], zero1_v)
        pltpu.sync_copy(zero1_v, den_hbm.at[pl.ds(base + t * CB, CB)])
    pltpu.sync_copy(den_sh.at[pl.ds(base + _NFULL * CB, _REM)],
                    zero1_v.at[pl.ds(0, _REM)])
    pltpu.sync_copy(zero1_v.at[pl.ds(0, _REM)],
                    den_hbm.at[pl.ds(base + _NFULL * CB, _REM)])


def _writeback_div(acc_sh, den_sh, x_hbm, rows, den_v, sid, out_base, sems):
    base = sid * _TPR
    hs = [None] * Q

    def div_chunk(b, n):
        def dg(g, _):
            dv = den_v[pl.ds(g * 16, 16)]
            for lane in range(16):
                dd = dv[lane] + 1e-16
                rr = g * 16 + lane
                y0 = rows[b][rr, pl.ds(0, 16)] / dd
                y1 = rows[b][rr, pl.ds(16, 16)] / dd
                rows[b][rr, pl.ds(0, 16)] = jnp.where(y0 >= 0, y0, 0.01 * y0)
                rows[b][rr, pl.ds(16, 16)] = jnp.where(y1 >= 0, y1, 0.01 * y1)
            return 0

        lax.fori_loop(0, n // 16, dg, 0)

    for t in range(_NFULL):
        b = t % Q
        if hs[b] is not None:
            hs[b].wait()
        pltpu.sync_copy(acc_sh.at[pl.ds(base + t * CB, CB)], rows[b])
        pltpu.sync_copy(den_sh.at[pl.ds(base + t * CB, CB)], den_v)
        div_chunk(b, CB)
        hs[b] = pltpu.async_copy(rows[b],
                                 x_hbm.at[pl.ds(out_base + base + t * CB, CB)],
                                 sems[b])
    b = _NFULL % Q
    if hs[b] is not None:
        hs[b].wait()
        hs[b] = None
    pltpu.sync_copy(acc_sh.at[pl.ds(base + _NFULL * CB, _REM)],
                    rows[b].at[pl.ds(0, _REM)])
    pltpu.sync_copy(den_sh.at[pl.ds(base + _NFULL * CB, _REM)],
                    den_v.at[pl.ds(0, _REM)])
    div_chunk(b, _REM)
    hs[b] = pltpu.async_copy(rows[b].at[pl.ds(0, _REM)],
                             x_hbm.at[pl.ds(out_base + base + _NFULL * CB,
                                            _REM)], sems[b])
    for h_ in hs:
        if h_ is not None:
            h_.wait()


def _sc_gat_body(src_hbm, dst_hbm, as_hbm, ad_hbm, ub_hbm, h_hbm,
                 x_out_hbm,
                 idxq, idxo, dstq, asg, adg, wq, r0, r1, ub_v, zero1_v,
                 acc_sh, den_sh,
                 sa0, sa1, sr0, sr1, ss):
    rows = (r0, r1)
    sas = (sa0, sa1)
    srs = (sr0, sr1)
    cid = lax.axis_index("c")
    sid = lax.axis_index("s")

    _zero_rows(r0, CB)
    _zero_1d(zero1_v, CB)
    _zero_spmem(acc_sh, den_sh, r0, zero1_v, sid, ss)
    pltpu.sync_copy(ub_hbm, ub_v)
    plsc.subcore_barrier()

    ubv = ub_v[...]
    off = cid * NP

    def body(j, _):
        row0 = sid * (EPT // CB) + j * Q
        pltpu.sync_copy(src_hbm.at[pl.ds(row0, Q)], idxq)
        pltpu.sync_copy(dst_hbm.at[pl.ds(row0, Q)], dstq)
        ghs = []
        for q in range(Q):
            ghs.append((pltpu.async_copy(as_hbm.at[idxq.at[q]], asg.at[q],
                                         sas[q]),
                        pltpu.async_copy(ad_hbm.at[dstq.at[q]], adg.at[q],
                                         sas[q])))
        for q in range(Q):
            def ib(g, _, q=q):
                sl = pl.ds(g * 16, 16)
                idxo[q, sl] = idxq[q, sl] + off
                return 0

            lax.fori_loop(0, CB // 16, ib, 0)
        rhs = []
        for q in range(Q):
            rhs.append(pltpu.async_copy(h_hbm.at[idxo.at[q]], rows[q], srs[q]))
        shs = []
        for q in range(Q):
            ghs[q][0].wait()
            ghs[q][1].wait()

            def wb(g, _, q=q):
                sl = pl.ds(g * 16, 16)
                a = asg[q, sl] + adg[q, sl]
                alpha = jnp.where(a >= 0, a, 0.2 * a)
                wq[q, sl] = jnp.exp(alpha - ubv)
                return 0

            lax.fori_loop(0, CB // 16, wb, 0)
            rhs[q].wait()

            def sb(g, _, q=q):
                wv = wq[q, pl.ds(g * 16, 16)]
                for lane in range(16):
                    ws = wv[lane]
                    rr = g * 16 + lane
                    rows[q][rr, pl.ds(0, 16)] = rows[q][rr, pl.ds(0, 16)] * ws
                    rows[q][rr, pl.ds(16, 16)] = rows[q][rr, pl.ds(16, 16)] * ws
                return 0

            lax.fori_loop(0, CB // 16, sb, 0)
            shs.append(pltpu.async_copy(rows[q], acc_sh.at[dstq.at[q]], ss,
                                        add=True))
            shs.append(pltpu.async_copy(wq.at[q], den_sh.at[dstq.at[q]], ss,
                                        add=True))

        for h_ in shs:
            h_.wait()
        return 0

    lax.fori_loop(0, NBODY, body, 0)
    plsc.subcore_barrier()
    _writeback_div(acc_sh, den_sh, x_out_hbm, rows, zero1_v, sid, cid * NP, srs)


def _sc_sage_body(src_hbm, dst_hbm, x_hbm, *args, with_cnt):
    if with_cnt:
        (acc_hbm, cnt_hbm, idxq, dstq, ones_v, r0, r1, zero1_v,
         acc_sh, den_sh, sr0, sr1, ss) = args
    else:
        (acc_hbm, idxq, dstq, ones_v, r0, r1, zero1_v,
         acc_sh, den_sh, sr0, sr1, ss) = args
    rows = (r0, r1)
    srs = (sr0, sr1)
    cid = lax.axis_index("c")
    sid = lax.axis_index("s")

    _zero_rows(r0, CB)
    _zero_1d(zero1_v, CB)
    _zero_spmem(acc_sh, den_sh if with_cnt else None, r0, zero1_v, sid, ss)
    if with_cnt:
        one = jnp.ones((16,), jnp.float32)
        for q in range(Q):
            def ob(g, _, q=q):
                ones_v[q, pl.ds(g * 16, 16)] = one
                return 0

            lax.fori_loop(0, CB // 16, ob, 0)
    plsc.subcore_barrier()

    off = cid * NP

    def body(j, _):
        row0 = sid * (EPT // CB) + j * Q
        pltpu.sync_copy(src_hbm.at[pl.ds(row0, Q)], idxq)
        pltpu.sync_copy(dst_hbm.at[pl.ds(row0, Q)], dstq)
        for q in range(Q):
            def ib(g, _, q=q):
                sl = pl.ds(g * 16, 16)
                idxq[q, sl] = idxq[q, sl] + off
                return 0

            lax.fori_loop(0, CB // 16, ib, 0)
        rhs = []
        for q in range(Q):
            rhs.append(pltpu.async_copy(x_hbm.at[idxq.at[q]], rows[q], srs[q]))
        shs = []
        for q in range(Q):
            rhs[q].wait()
            shs.append(pltpu.async_copy(rows[q], acc_sh.at[dstq.at[q]], ss,
                                        add=True))
        if with_cnt:
            @pl.when(cid == 1)
            def _():
                for q in range(Q):
                    pltpu.sync_copy(ones_v.at[q], den_sh.at[dstq.at[q]],
                                    add=True)
        for h_ in shs:
            h_.wait()
        return 0

    lax.fori_loop(0, NBODY, body, 0)
    plsc.subcore_barrier()
    _writeback(acc_sh, acc_hbm, rows, sid, cid * NP, srs)

    if with_cnt:
        @pl.when(cid == 1)
        def _():
            _writeback_1d(den_sh, cnt_hbm, zero1_v, sid, ss)


def _sc_head_body(upn_idx_hbm, vid_idx_hbm, res_hbm, vidf_hbm,
                  upn_out, vid_out, idx_v, vidx_v, rows_v, vrow_v, sem):
    cid = lax.axis_index("c")
    sid = lax.axis_index("s")
    w = sid * 2 + cid

    pltpu.sync_copy(upn_idx_hbm.at[pl.ds(w * 3, 3)], idx_v)
    cps = []
    for k2 in range(3):
        cps.append(pltpu.async_copy(res_hbm.at[idx_v.at[k2]],
                                    rows_v.at[pl.ds(k2 * 128, 128)], sem))
    pltpu.sync_copy(vid_idx_hbm.at[pl.ds(w, 1)], vidx_v)
    cps.append(pltpu.async_copy(vidf_hbm.at[vidx_v.at[0]], vrow_v, sem))
    for cp in cps:
        cp.wait()
    pltpu.sync_copy(rows_v, upn_out.at[pl.ds(w * 384, 384)])
    pltpu.sync_copy(vrow_v, vid_out.at[pl.ds(w * 128, 128)])


def _sc_mesh():
    return plsc.VectorSubcoreMesh(core_axis_name="c", subcore_axis_name="s")


def _sc_gat(src2, dst2, a_s, a_d, ub16, h_flat):
    f32 = jnp.float32
    call = pl.kernel(
        _sc_gat_body,
        out_type=[
            jax.ShapeDtypeStruct((2 * NP, H), f32),
        ],
        mesh=_sc_mesh(),
        compiler_params=pltpu.CompilerParams(use_tc_tiling_on_sc=False),
        scratch_types=[
            pltpu.VMEM((Q, CB), jnp.int32),
            pltpu.VMEM((Q, CB), jnp.int32),
            pltpu.VMEM((Q, CB), jnp.int32),
            pltpu.VMEM((Q, CB), f32),
            pltpu.VMEM((Q, CB), f32),
            pltpu.VMEM((Q, CB), f32),
            pltpu.VMEM((CB, H), f32),
            pltpu.VMEM((CB, H), f32),
            pltpu.VMEM((16,), f32),
            pltpu.VMEM((CB,), f32),
            pltpu.VMEM_SHARED((NP, H), f32),
            pltpu.VMEM_SHARED((NP,), f32),
            pltpu.SemaphoreType.DMA,
            pltpu.SemaphoreType.DMA,
            pltpu.SemaphoreType.DMA,
            pltpu.SemaphoreType.DMA,
            pltpu.SemaphoreType.DMA,
        ],
    )
    return call(src2, dst2, a_s, a_d, ub16, h_flat)


def _sc_sage(src2, dst2, x_flat, with_cnt):
    f32 = jnp.float32
    out_type = [jax.ShapeDtypeStruct((2 * NP, H), f32)]
    if with_cnt:
        out_type.append(jax.ShapeDtypeStruct((NP,), f32))
    call = pl.kernel(
        functools.partial(_sc_sage_body, with_cnt=with_cnt),
        out_type=out_type,
        mesh=_sc_mesh(),
        compiler_params=pltpu.CompilerParams(use_tc_tiling_on_sc=False),
        scratch_types=[
            pltpu.VMEM((Q, CB), jnp.int32),
            pltpu.VMEM((Q, CB), jnp.int32),
            pltpu.VMEM((Q, CB), f32),
            pltpu.VMEM((CB, H), f32),
            pltpu.VMEM((CB, H), f32),
            pltpu.VMEM((CB,), f32),
            pltpu.VMEM_SHARED((NP, H), f32),
            pltpu.VMEM_SHARED((NP,), f32),
            pltpu.SemaphoreType.DMA,
            pltpu.SemaphoreType.DMA,
            pltpu.SemaphoreType.DMA,
        ],
    )
    return call(src2, dst2, x_flat)


def _sc_head(upn_idx2, vid_idx2, result, video_features):
    f32 = jnp.float32
    call = pl.kernel(
        _sc_head_body,
        out_type=[
            jax.ShapeDtypeStruct((3 * B, D), f32),
            jax.ShapeDtypeStruct((B, 128), f32),
        ],
        mesh=_sc_mesh(),
        compiler_params=pltpu.CompilerParams(use_tc_tiling_on_sc=False),
        scratch_types=[
            pltpu.VMEM((3, 128), jnp.int32),
            pltpu.VMEM((1, 128), jnp.int32),
            pltpu.VMEM((384, D), f32),
            pltpu.VMEM((128, 128), f32),
            pltpu.SemaphoreType.DMA,
        ],
    )
    return call(upn_idx2, vid_idx2, result, video_features)


def _pad_edges(ei):
    pad = E_PAD - E
    src = jnp.concatenate([ei[0], jnp.zeros((pad,), jnp.int32)])
    dst = jnp.concatenate([ei[1], jnp.full((pad,), TRASH, jnp.int32)])
    return src.reshape(E_PAD // CB, CB), dst.reshape(E_PAD // CB, CB)


def kernel(item, video_features, u_h_embedding, uh_edge_index, v_uh_edge_index,
           trans_W, trans_b, gat1_W, gat1_att, sage2_W, sage2_b,
           gat3_W, gat3_att, sage4_W, sage4_b, uv_W, uv_b, uh_W, uh_b):
    f32 = jnp.float32
    src_g, dst_g = _pad_edges(v_uh_edge_index)
    src_s, dst_s = _pad_edges(uh_edge_index)

    att1 = jnp.stack([gat1_att[:D], gat1_att[D:]])
    att3 = jnp.stack([gat3_att[:D], gat3_att[D:]])

    h1 = _gat1_prep(u_h_embedding, video_features, trans_W, trans_b, gat1_W)
    as1, ad1, mxs1, mxd1 = _attn_prep(h1, att1)
    ub1 = _leaky(mxs1[0, 0] + mxd1[0, 0], 0.2)
    (x1,) = _sc_gat(src_g, dst_g, as1.reshape(NP), ad1.reshape(NP),
                    jnp.full((16,), ub1, f32), h1.reshape(2 * NP, H))

    s2, cnt = _sc_sage(src_s, dst_s, x1, with_cnt=True)
    cnt = cnt.reshape(NP, 1)

    h3, as3, ad3, mxs3, mxd3 = _sage_gat_prep(s2.reshape(2, NP, H), cnt,
                                              sage2_W, sage2_b, gat3_W, att3)
    ub3 = _leaky(mxs3[0, 0] + mxd3[0, 0], 0.2)
    (x3,) = _sc_gat(src_g, dst_g, as3.reshape(NP), ad3.reshape(NP),
                    jnp.full((16,), ub3, f32), h3.reshape(2 * NP, H))

    (s4,) = _sc_sage(src_s, dst_s, x3, with_cnt=False)
    result = _sage_final(s4.reshape(2, NP, H), cnt, sage4_W, sage4_b)

    upn_idx = jnp.concatenate([item[:, 0], item[:, 2], item[:, 3]])
    upn_idx2 = upn_idx.reshape(3 * B // 128, 128)
    vid_idx2 = (item[:, 1] - N_UH).reshape(B // 128, 128)
    upn, vid = _sc_head(upn_idx2, vid_idx2, result, video_features)
    return _head(upn, vid, trans_W, trans_b, uv_W, uv_b, uh_W, uh_b)
